# Initial kernel scaffold; baseline (speedup 1.0000x reference)
import functools

import jax
import jax.numpy as jnp
from jax import lax
from jax.experimental import pallas as pl
from jax.experimental.pallas import tpu as pltpu
from jax.experimental.pallas import tpu_sc as plsc

_NS = 16
_K = 256


def _round_up(x, m):
    return ((x + m - 1) // m) * m


def _pick_bn(n, cap=2048):
    bn = 8
    for d in range(8, min(n, cap) + 1, 8):
        if n % d == 0:
            bn = d
    return bn


def _proj_tc(parts, bias_row, wst, n, relu):
    j_n = wst.shape[0]
    bn = _pick_bn(n)
    nb = n // bn
    npart = len(parts)

    def body(*refs):
        part_refs = refs[:npart]
        bias_ref = refs[npart]
        w_ref = refs[npart + 1]
        out_ref = refs[npart + 2]
        x = part_refs[0][...]
        for p in part_refs[1:]:
            x = x + p[...]
        x = x + bias_ref[...]
        if relu:
            x = jnp.maximum(x, 0.0)
        out_ref[0] = jnp.dot(x, w_ref[0], preferred_element_type=jnp.float32)

    in_specs = (
        [pl.BlockSpec((bn, 128), lambda i, j: (i, 0)) for _ in parts]
        + [pl.BlockSpec((1, 128), lambda i, j: (0, 0))]
        + [pl.BlockSpec((1, 128, 128), lambda i, j: (j, 0, 0))]
    )
    return pl.pallas_call(
        body,
        grid=(nb, j_n),
        in_specs=in_specs,
        out_specs=pl.BlockSpec((1, bn, 128), lambda i, j: (j, i, 0)),
        out_shape=jax.ShapeDtypeStruct((j_n, n, 128), jnp.float32),
    )(*parts, bias_row, wst)


def _finalize_tc(parts, bias_row, n_pad):
    bn = 1024
    npart = len(parts)

    def body(*refs):
        x = refs[0][...]
        for p in refs[1:npart]:
            x = x + p[...]
        x = x + refs[npart][...]
        refs[npart + 1][...] = jnp.maximum(x, 0.0)

    in_specs = [pl.BlockSpec((bn, 128), lambda i: (i, 0)) for _ in parts] + [
        pl.BlockSpec((1, 128), lambda i: (0, 0))
    ]
    return pl.pallas_call(
        body,
        grid=(n_pad // bn,),
        in_specs=in_specs,
        out_specs=pl.BlockSpec((bn, 128), lambda i: (i, 0)),
        out_shape=jax.ShapeDtypeStruct((n_pad, 128), jnp.float32),
    )(*parts, bias_row)


def _mlp_tc(g1, g2, g3, w1a, w1b, w1c, b1, w2, b2, w3, b3):
    b_n = g1.shape[0]
    bm = 512

    def body(g1r, g2r, g3r, w1ar, w1br, w1cr, b1r, w2r, b2r, w3r, b3r, outr):
        h = jnp.dot(g1r[...], w1ar[...], preferred_element_type=jnp.float32)
        h = h + jnp.dot(g2r[...], w1br[...], preferred_element_type=jnp.float32)
        h = h + jnp.dot(g3r[...], w1cr[...], preferred_element_type=jnp.float32)
        h = jnp.maximum(h + b1r[...], 0.0)
        h = jnp.maximum(
            jnp.dot(h, w2r[...], preferred_element_type=jnp.float32) + b2r[...],
            0.0,
        )
        outr[...] = (
            jnp.dot(h, w3r[...], preferred_element_type=jnp.float32) + b3r[...]
        )

    def full(shape):
        return pl.BlockSpec(shape, lambda i: tuple(0 for _ in shape))

    return pl.pallas_call(
        body,
        grid=(b_n // bm,),
        in_specs=[
            pl.BlockSpec((bm, 128), lambda i: (i, 0)),
            pl.BlockSpec((bm, 128), lambda i: (i, 0)),
            pl.BlockSpec((bm, 128), lambda i: (i, 0)),
            full((128, 768)),
            full((128, 768)),
            full((128, 768)),
            full((1, 768)),
            full((768, 256)),
            full((1, 256)),
            full((256, 128)),
            full((1, 128)),
        ],
        out_specs=pl.BlockSpec((bm, 128), lambda i: (i, 0)),
        out_shape=jax.ShapeDtypeStruct((b_n, 128), jnp.float32),
    )(g1, g2, g3, w1a, w1b, w1c, b1, w2, b2, w3, b3)


def _gat_edge_sc(h2, a_src, a_dst, src_e, dst_e, n_dst, n_pad):
    n_src = a_src.shape[0]
    e_pad = src_e.shape[0]
    cw = e_pad // (_NS * _K)
    r16 = n_pad // _NS
    cr = 64
    mesh = plsc.VectorSubcoreMesh(core_axis_name="c", subcore_axis_name="s")

    @functools.partial(
        pl.kernel,
        out_type=jax.ShapeDtypeStruct((n_pad, 128), jnp.float32),
        mesh=mesh,
        scratch_types=[
            pltpu.VMEM((n_src,), jnp.float32),
            pltpu.VMEM((n_dst,), jnp.float32),
            pltpu.VMEM((_K,), jnp.int32),
            pltpu.VMEM((_K,), jnp.int32),
            pltpu.VMEM((_K,), jnp.int32),
            pltpu.VMEM((_K,), jnp.float32),
            pltpu.VMEM((_K, 64), jnp.float32),
            pltpu.VMEM((_K, 80), jnp.float32),
            pltpu.VMEM((8, 80), jnp.float32),
            pltpu.VMEM((cr, 80), jnp.float32),
            pltpu.VMEM((cr, 64), jnp.float32),
            pltpu.VMEM_SHARED((n_pad, 80), jnp.float32),
            pltpu.SemaphoreType.DMA,
        ],
    )
    def k(h2_hbm, asrc_hbm, adst_hbm, src_hbm, dst_hbm, out_hbm,
          asrc_v, adst_v, sbuf, dbuf, gidx, wbuf, rowbuf, widebuf, zbuf,
          drainbuf, obuf, num_sh, sem):
        c = lax.axis_index("c")
        s = lax.axis_index("s")
        zero16 = jnp.zeros((16,), jnp.float32)
        lanes = jnp.arange(16, dtype=jnp.int32)
        for r in range(8):
            for q in range(5):
                zbuf[r, pl.ds(q * 16, 16)] = zero16

        def zbody(i, carry):
            pltpu.sync_copy(zbuf, num_sh.at[pl.ds(s * r16 + i * 8, 8)])
            return carry

        lax.fori_loop(0, r16 // 8, zbody, 0)
        pltpu.sync_copy(asrc_hbm, asrc_v)
        pltpu.sync_copy(adst_hbm, adst_v)
        plsc.subcore_barrier()

        def chunk_body(i, carry):
            base = (s * cw + i) * _K
            pltpu.sync_copy(src_hbm.at[pl.ds(base, _K)], sbuf)
            pltpu.sync_copy(dst_hbm.at[pl.ds(base, _K)], dbuf)

            def wbody(j, carry2):
                sl = pl.ds(j * 16, 16)
                sv = sbuf[sl]
                dv = jnp.minimum(dbuf[sl], n_dst - 1)
                al = plsc.load_gather(asrc_v, [sv]) + plsc.load_gather(
                    adst_v, [dv]
                )
                al = jnp.where(al >= 0.0, al, 0.2 * al)
                wbuf[sl] = jnp.exp(al)
                gidx[sl] = sv * 2 + c
                return carry2

            lax.fori_loop(0, _K // 16, wbody, 0)
            pltpu.async_copy(h2_hbm.at[gidx], rowbuf, sem).wait()

            def sbody(j, carry2):
                for l in range(16):
                    e = j * 16 + l
                    wb = plsc.load_gather(wbuf, [jnp.full((16,), e, jnp.int32)])
                    for q in range(4):
                        ql = pl.ds(q * 16, 16)
                        widebuf[e, ql] = rowbuf[e, ql] * wb
                    widebuf[e, pl.ds(64, 16)] = jnp.where(lanes == 0, wb, 0.0)
                return carry2

            lax.fori_loop(0, _K // 16, sbody, 0)
            pltpu.sync_copy(widebuf, num_sh.at[dbuf], add=True)
            return carry

        lax.fori_loop(0, cw, chunk_body, 0)
        plsc.subcore_barrier()

        def drain_body(i, carry):
            r0 = s * r16 + i * cr
            pltpu.sync_copy(num_sh.at[pl.ds(r0, cr)], drainbuf)
            for r in range(cr):
                den = plsc.load_gather(
                    drainbuf,
                    [jnp.full((16,), r, jnp.int32),
                     jnp.full((16,), 64, jnp.int32)],
                )
                m = den > 0.0
                for q in range(4):
                    ql = pl.ds(q * 16, 16)
                    obuf[r, ql] = jnp.where(m, drainbuf[r, ql] / den, 0.0)
            pltpu.sync_copy(obuf, out_hbm.at[pl.ds(r0, cr), pl.ds(c * 64, 64)])
            return carry

        lax.fori_loop(0, r16 // cr, drain_body, 0)

    return k(h2, a_src, a_dst, src_e, dst_e)


def _readout_sc(hd, hc, drug1, drug2, cell):
    b_n = drug1.shape[0]
    rb = b_n // 32
    mesh = plsc.VectorSubcoreMesh(core_axis_name="c", subcore_axis_name="s")
    out_t = jax.ShapeDtypeStruct((b_n, 128), jnp.float32)

    @functools.partial(
        pl.kernel,
        out_type=(out_t, out_t, out_t),
        mesh=mesh,
        scratch_types=[
            pltpu.VMEM((rb,), jnp.int32),
            pltpu.VMEM((rb, 128), jnp.float32),
            pltpu.SemaphoreType.DMA,
        ],
    )
    def k(hd_hbm, hc_hbm, d1_hbm, d2_hbm, cl_hbm, o1, o2, o3, idx_v, buf, sem):
        wid = lax.axis_index("s") * 2 + lax.axis_index("c")
        base = wid * rb
        for idx_hbm, tab_hbm, out_hbm in (
            (d1_hbm, hd_hbm, o1),
            (d2_hbm, hd_hbm, o2),
            (cl_hbm, hc_hbm, o3),
        ):
            pltpu.sync_copy(idx_hbm.at[pl.ds(base, rb)], idx_v)
            pltpu.async_copy(tab_hbm.at[idx_v], buf, sem).wait()
            pltpu.sync_copy(buf, out_hbm.at[pl.ds(base, rb)])

    return k(hd, hc, drug1, drug2, cell)


def _pad_edges(src, dst, n_dst):
    e = src.shape[0]
    e_pad = _round_up(e, _NS * _K)
    pad = e_pad - e
    src = jnp.concatenate([src, jnp.zeros((pad,), jnp.int32)])
    dst = jnp.concatenate([dst, jnp.full((pad,), n_dst, jnp.int32)])
    return src, dst


def _score_cols(w_l, specs):
    cols = [w_l[r] @ v[r] for (r, v) in specs]
    g = jnp.stack(cols, axis=1)
    return jnp.pad(g, ((0, 0), (0, 128 - g.shape[1])))


def kernel(x_drug, x_protein, x_cell, edge_index_dd, edge_index_dp,
           edge_index_rev_dp, edge_index_pp, edge_index_cp, edge_index_rev_cp,
           drug1, drug2, cell, drug_table, protein_table, cell_table,
           W0, as0, ad0, b0, W1, as1, ad1, b1, cW1, cb1, cW2, cb2, cW3, cb3):
    nd = drug_table.shape[0]
    np_ = protein_table.shape[0]
    nc = cell_table.shape[0]
    pad_d = _round_up(nd + 1, 1024)
    pad_p = _round_up(np_ + 1, 1024)
    pad_c = _round_up(nc + 1, 1024)

    hd0 = jnp.take(drug_table, x_drug, axis=0)
    hp0 = jnp.take(protein_table, x_protein, axis=0)
    hc0 = jnp.take(cell_table, x_cell, axis=0)

    ar_d = jnp.arange(nd, dtype=jnp.int32)
    ar_p = jnp.arange(np_, dtype=jnp.int32)
    s_dd, d_dd = _pad_edges(
        jnp.concatenate([edge_index_dd[0], ar_d]),
        jnp.concatenate([edge_index_dd[1], ar_d]), nd)
    s_dp, d_dp = _pad_edges(edge_index_dp[0], edge_index_dp[1], np_)
    s_rdp, d_rdp = _pad_edges(edge_index_rev_dp[0], edge_index_rev_dp[1], nd)
    s_pp, d_pp = _pad_edges(
        jnp.concatenate([edge_index_pp[0], ar_p]),
        jnp.concatenate([edge_index_pp[1], ar_p]), np_)
    s_cp, d_cp = _pad_edges(edge_index_cp[0], edge_index_cp[1], np_)
    s_rcp, d_rcp = _pad_edges(edge_index_rev_cp[0], edge_index_rev_cp[1], nc)

    zbias = jnp.zeros((1, 128), jnp.float32)

    gd0 = _score_cols(W0, [(0, as0), (1, as0), (0, ad0), (2, ad0)])
    gp0 = _score_cols(
        W0, [(2, as0), (3, as0), (5, as0), (1, ad0), (3, ad0), (4, ad0)])
    gc0 = _score_cols(W0, [(4, as0), (5, ad0)])
    yd = _proj_tc([hd0], zbias, jnp.stack([W0[0], W0[1], gd0]), nd, False)
    yp = _proj_tc([hp0], zbias,
                  jnp.stack([W0[2], W0[3], W0[5], gp0]), np_, False)
    yc = _proj_tc([hc0], zbias, jnp.stack([W0[4], gc0]), nc, False)

    sd = yd[2]
    sp = yp[3]
    sc = yc[1]

    def h2(y):
        return y.reshape(2 * y.shape[0], 64)

    od_dd = _gat_edge_sc(h2(yd[0]), sd[:, 0], sd[:, 2], s_dd, d_dd, nd, pad_d)
    op_dp = _gat_edge_sc(h2(yd[1]), sd[:, 1], sp[:, 3], s_dp, d_dp, np_, pad_p)
    od_rdp = _gat_edge_sc(
        h2(yp[0]), sp[:, 0], sd[:, 3], s_rdp, d_rdp, nd, pad_d)
    op_pp = _gat_edge_sc(h2(yp[1]), sp[:, 1], sp[:, 4], s_pp, d_pp, np_, pad_p)
    op_cp = _gat_edge_sc(h2(yc[0]), sc[:, 0], sp[:, 5], s_cp, d_cp, np_, pad_p)
    oc_rcp = _gat_edge_sc(
        h2(yp[2]), sp[:, 2], sc[:, 1], s_rcp, d_rcp, nc, pad_c)

    gd1 = _score_cols(W1, [(0, as1), (0, ad1), (2, ad1)])
    gp1 = _score_cols(W1, [(2, as1), (5, as1)])
    gc1 = _score_cols(W1, [(5, ad1)])
    bias_d = (b0[0] + b0[2]).reshape(1, 128)
    bias_p = (b0[1] + b0[3] + b0[4]).reshape(1, 128)
    bias_c = b0[5].reshape(1, 128)
    yd1 = _proj_tc([od_dd, od_rdp], bias_d, jnp.stack([W1[0], gd1]), nd, True)
    yp1 = _proj_tc([op_dp, op_pp, op_cp], bias_p,
                   jnp.stack([W1[2], W1[5], gp1]), np_, True)
    yc1 = _proj_tc([oc_rcp], bias_c, jnp.stack([gc1]), nc, True)

    sd1 = yd1[1]
    sp1 = yp1[2]
    sc1 = yc1[0]
    od_dd1 = _gat_edge_sc(
        h2(yd1[0]), sd1[:, 0], sd1[:, 1], s_dd, d_dd, nd, pad_d)
    od_rdp1 = _gat_edge_sc(
        h2(yp1[0]), sp1[:, 0], sd1[:, 2], s_rdp, d_rdp, nd, pad_d)
    oc_rcp1 = _gat_edge_sc(
        h2(yp1[1]), sp1[:, 1], sc1[:, 0], s_rcp, d_rcp, nc, pad_c)

    hd_fin = _finalize_tc(
        [od_dd1, od_rdp1], (b1[0] + b1[2]).reshape(1, 128), pad_d)
    hc_fin = _finalize_tc([oc_rcp1], b1[5].reshape(1, 128), pad_c)
    g1, g2, g3 = _readout_sc(hd_fin, hc_fin, drug1, drug2, cell)

    w3p = jnp.pad(cW3, ((0, 0), (0, 126)))
    b3p = jnp.pad(cb3, (0, 126)).reshape(1, 128)
    out = _mlp_tc(g1, g2, g3, cW1[:128], cW1[128:256], cW1[256:384],
                  cb1.reshape(1, 768), cW2, cb2.reshape(1, 256), w3p, b3p)
    return out[:, :2]


# speedup vs baseline: 8.8527x; 8.8527x over previous
import functools

import jax
import jax.numpy as jnp
from jax import lax
from jax.experimental import pallas as pl
from jax.experimental.pallas import tpu as pltpu
from jax.experimental.pallas import tpu_sc as plsc

_NS = 16
_K = 128


def _round_up(x, m):
    return ((x + m - 1) // m) * m


def _pick_bn(n, cap=2048):
    bn = 8
    for d in range(8, min(n, cap) + 1, 8):
        if n % d == 0:
            bn = d
    return bn


def _proj_tc(parts, bias_row, wst, n, relu):
    j_n = wst.shape[0]
    bn = _pick_bn(n)
    nb = n // bn
    npart = len(parts)

    def body(*refs):
        part_refs = refs[:npart]
        bias_ref = refs[npart]
        w_ref = refs[npart + 1]
        out_ref = refs[npart + 2]
        x = part_refs[0][...]
        for p in part_refs[1:]:
            x = x + p[...]
        x = x + bias_ref[...]
        if relu:
            x = jnp.maximum(x, 0.0)
        out_ref[0] = jnp.dot(x, w_ref[0], preferred_element_type=jnp.float32)

    in_specs = (
        [pl.BlockSpec((bn, 128), lambda i, j: (i, 0)) for _ in parts]
        + [pl.BlockSpec((1, 128), lambda i, j: (0, 0))]
        + [pl.BlockSpec((1, 128, 128), lambda i, j: (j, 0, 0))]
    )
    return pl.pallas_call(
        body,
        grid=(nb, j_n),
        in_specs=in_specs,
        out_specs=pl.BlockSpec((1, bn, 128), lambda i, j: (j, i, 0)),
        out_shape=jax.ShapeDtypeStruct((j_n, n, 128), jnp.float32),
    )(*parts, bias_row, wst)


def _finalize_tc(parts, bias_row, n_pad):
    bn = 1024
    npart = len(parts)

    def body(*refs):
        x = refs[0][...]
        for p in refs[1:npart]:
            x = x + p[...]
        x = x + refs[npart][...]
        refs[npart + 1][...] = jnp.maximum(x, 0.0)

    in_specs = [pl.BlockSpec((bn, 128), lambda i: (i, 0)) for _ in parts] + [
        pl.BlockSpec((1, 128), lambda i: (0, 0))
    ]
    return pl.pallas_call(
        body,
        grid=(n_pad // bn,),
        in_specs=in_specs,
        out_specs=pl.BlockSpec((bn, 128), lambda i: (i, 0)),
        out_shape=jax.ShapeDtypeStruct((n_pad, 128), jnp.float32),
    )(*parts, bias_row)


def _mlp_tc(g1, g2, g3, w1a, w1b, w1c, b1, w2, b2, w3, b3):
    b_n = g1.shape[0]
    bm = 512

    def body(g1r, g2r, g3r, w1ar, w1br, w1cr, b1r, w2r, b2r, w3r, b3r, outr):
        h = jnp.dot(g1r[...], w1ar[...], preferred_element_type=jnp.float32)
        h = h + jnp.dot(g2r[...], w1br[...], preferred_element_type=jnp.float32)
        h = h + jnp.dot(g3r[...], w1cr[...], preferred_element_type=jnp.float32)
        h = jnp.maximum(h + b1r[...], 0.0)
        h = jnp.maximum(
            jnp.dot(h, w2r[...], preferred_element_type=jnp.float32) + b2r[...],
            0.0,
        )
        outr[...] = (
            jnp.dot(h, w3r[...], preferred_element_type=jnp.float32) + b3r[...]
        )

    def full(shape):
        return pl.BlockSpec(shape, lambda i: tuple(0 for _ in shape))

    return pl.pallas_call(
        body,
        grid=(b_n // bm,),
        in_specs=[
            pl.BlockSpec((bm, 128), lambda i: (i, 0)),
            pl.BlockSpec((bm, 128), lambda i: (i, 0)),
            pl.BlockSpec((bm, 128), lambda i: (i, 0)),
            full((128, 768)),
            full((128, 768)),
            full((128, 768)),
            full((1, 768)),
            full((768, 256)),
            full((1, 256)),
            full((256, 128)),
            full((1, 128)),
        ],
        out_specs=pl.BlockSpec((bm, 128), lambda i: (i, 0)),
        out_shape=jax.ShapeDtypeStruct((b_n, 128), jnp.float32),
    )(g1, g2, g3, w1a, w1b, w1c, b1, w2, b2, w3, b3)


def _gat_edge_sc(h2, a_src, a_dst, src_e, dst_e, n_dst, n_pad):
    e_pad = src_e.shape[0]
    cw = e_pad // (_NS * _K)
    r16 = n_pad // _NS
    cr = 16
    mesh = plsc.VectorSubcoreMesh(core_axis_name="c", subcore_axis_name="s")

    @functools.partial(
        pl.kernel,
        out_type=jax.ShapeDtypeStruct((n_pad, 128), jnp.float32),
        mesh=mesh,
        compiler_params=pltpu.CompilerParams(
            use_tc_tiling_on_sc=False, needs_layout_passes=False),
        scratch_types=[
            pltpu.VMEM((_K,), jnp.int32),
            pltpu.VMEM((_K,), jnp.int32),
            pltpu.VMEM((_K,), jnp.int32),
            pltpu.VMEM((_K,), jnp.int32),
            pltpu.VMEM((_K,), jnp.float32),
            pltpu.VMEM((_K,), jnp.float32),
            pltpu.VMEM((_K,), jnp.float32),
            pltpu.VMEM((_K, 64), jnp.float32),
            pltpu.VMEM((_K, 80), jnp.float32),
            pltpu.VMEM((8, 80), jnp.float32),
            pltpu.VMEM((cr, 80), jnp.float32),
            pltpu.VMEM((cr, 64), jnp.float32),
            pltpu.VMEM_SHARED((n_pad, 80), jnp.float32),
            pltpu.SemaphoreType.DMA,
        ],
    )
    def k(h2_hbm, asrc_hbm, adst_hbm, src_hbm, dst_hbm, out_hbm,
          sbuf, dbuf, gidx, dcl, av, bv, wbuf, rowbuf, widebuf, zbuf,
          drainbuf, obuf, num_sh, sem):
        c = lax.axis_index("c")
        s = lax.axis_index("s")
        zero16 = jnp.zeros((16,), jnp.float32)
        lanes = jnp.arange(16, dtype=jnp.int32)
        for r in range(8):
            for q in range(5):
                zbuf[r, pl.ds(q * 16, 16)] = zero16

        def zbody(i, carry):
            pltpu.sync_copy(zbuf, num_sh.at[pl.ds(s * r16 + i * 8, 8)])
            return carry

        lax.fori_loop(0, r16 // 8, zbody, 0)
        plsc.subcore_barrier()

        def chunk_body(i, carry):
            base = (s * cw + i) * _K
            pltpu.sync_copy(src_hbm.at[pl.ds(base, _K)], sbuf)
            pltpu.sync_copy(dst_hbm.at[pl.ds(base, _K)], dbuf)

            def ibody(j, carry2):
                sl = pl.ds(j * 16, 16)
                gidx[sl] = sbuf[sl] * 2 + c
                dcl[sl] = jnp.minimum(dbuf[sl], n_dst - 1)
                return carry2

            lax.fori_loop(0, _K // 16, ibody, 0)
            d1 = pltpu.async_copy(asrc_hbm.at[sbuf], av, sem)
            d2 = pltpu.async_copy(adst_hbm.at[dcl], bv, sem)
            d3 = pltpu.async_copy(h2_hbm.at[gidx], rowbuf, sem)
            d1.wait()
            d2.wait()
            d3.wait()

            def sbody(j, carry2):
                sl = pl.ds(j * 16, 16)
                al = av[sl] + bv[sl]
                al = jnp.where(al >= 0.0, al, 0.2 * al)
                wbuf[sl] = jnp.exp(al)
                for l in range(16):
                    e = j * 16 + l
                    wb = plsc.load_gather(wbuf, [jnp.full((16,), e, jnp.int32)])
                    for q in range(4):
                        ql = pl.ds(q * 16, 16)
                        widebuf[e, ql] = rowbuf[e, ql] * wb
                    widebuf[e, pl.ds(64, 16)] = jnp.where(lanes == 0, wb, 0.0)
                return carry2

            lax.fori_loop(0, _K // 16, sbody, 0)
            pltpu.sync_copy(widebuf, num_sh.at[dbuf], add=True)
            return carry

        lax.fori_loop(0, cw, chunk_body, 0)
        plsc.subcore_barrier()

        def drain_body(i, carry):
            r0 = s * r16 + i * cr
            pltpu.sync_copy(num_sh.at[pl.ds(r0, cr)], drainbuf)
            for r in range(cr):
                den = plsc.load_gather(
                    drainbuf,
                    [jnp.full((16,), r, jnp.int32),
                     jnp.full((16,), 64, jnp.int32)],
                )
                m = den > 0.0
                for q in range(4):
                    ql = pl.ds(q * 16, 16)
                    obuf[r, ql] = jnp.where(m, drainbuf[r, ql] / den, 0.0)
            pltpu.sync_copy(obuf, out_hbm.at[pl.ds(r0, cr), pl.ds(c * 64, 64)])
            return carry

        lax.fori_loop(0, r16 // cr, drain_body, 0)

    return k(h2, a_src, a_dst, src_e, dst_e)


def _readout_sc(hd, hc, drug1, drug2, cell):
    b_n = drug1.shape[0]
    rb = b_n // 32
    mesh = plsc.VectorSubcoreMesh(core_axis_name="c", subcore_axis_name="s")
    out_t = jax.ShapeDtypeStruct((b_n, 128), jnp.float32)

    @functools.partial(
        pl.kernel,
        out_type=(out_t, out_t, out_t),
        mesh=mesh,
        compiler_params=pltpu.CompilerParams(use_tc_tiling_on_sc=False, needs_layout_passes=False),
        scratch_types=[
            pltpu.VMEM((rb,), jnp.int32),
            pltpu.VMEM((rb, 128), jnp.float32),
            pltpu.SemaphoreType.DMA,
        ],
    )
    def k(hd_hbm, hc_hbm, d1_hbm, d2_hbm, cl_hbm, o1, o2, o3, idx_v, buf, sem):
        wid = lax.axis_index("s") * 2 + lax.axis_index("c")
        base = wid * rb
        for idx_hbm, tab_hbm, out_hbm in (
            (d1_hbm, hd_hbm, o1),
            (d2_hbm, hd_hbm, o2),
            (cl_hbm, hc_hbm, o3),
        ):
            pltpu.sync_copy(idx_hbm.at[pl.ds(base, rb)], idx_v)
            pltpu.async_copy(tab_hbm.at[idx_v], buf, sem).wait()
            pltpu.sync_copy(buf, out_hbm.at[pl.ds(base, rb)])

    return k(hd, hc, drug1, drug2, cell)


def _pad_edges(src, dst, n_dst):
    e = src.shape[0]
    e_pad = _round_up(e, _NS * _K)
    pad = e_pad - e
    src = jnp.concatenate([src, jnp.zeros((pad,), jnp.int32)])
    dst = jnp.concatenate([dst, jnp.full((pad,), n_dst, jnp.int32)])
    return src, dst


def _score_cols(w_l, specs):
    cols = [w_l[r] @ v[r] for (r, v) in specs]
    g = jnp.stack(cols, axis=1)
    return jnp.pad(g, ((0, 0), (0, 128 - g.shape[1])))


def kernel(x_drug, x_protein, x_cell, edge_index_dd, edge_index_dp,
           edge_index_rev_dp, edge_index_pp, edge_index_cp, edge_index_rev_cp,
           drug1, drug2, cell, drug_table, protein_table, cell_table,
           W0, as0, ad0, b0, W1, as1, ad1, b1, cW1, cb1, cW2, cb2, cW3, cb3):
    nd = drug_table.shape[0]
    np_ = protein_table.shape[0]
    nc = cell_table.shape[0]
    pad_d = _round_up(nd + 1, 1024)
    pad_p = _round_up(np_ + 1, 256)
    pad_c = _round_up(nc + 1, 1024)

    hd0 = jnp.take(drug_table, x_drug, axis=0)
    hp0 = jnp.take(protein_table, x_protein, axis=0)
    hc0 = jnp.take(cell_table, x_cell, axis=0)

    ar_d = jnp.arange(nd, dtype=jnp.int32)
    ar_p = jnp.arange(np_, dtype=jnp.int32)
    s_dd, d_dd = _pad_edges(
        jnp.concatenate([edge_index_dd[0], ar_d]),
        jnp.concatenate([edge_index_dd[1], ar_d]), nd)
    s_dp, d_dp = _pad_edges(edge_index_dp[0], edge_index_dp[1], np_)
    s_rdp, d_rdp = _pad_edges(edge_index_rev_dp[0], edge_index_rev_dp[1], nd)
    s_pp, d_pp = _pad_edges(
        jnp.concatenate([edge_index_pp[0], ar_p]),
        jnp.concatenate([edge_index_pp[1], ar_p]), np_)
    s_cp, d_cp = _pad_edges(edge_index_cp[0], edge_index_cp[1], np_)
    s_rcp, d_rcp = _pad_edges(edge_index_rev_cp[0], edge_index_rev_cp[1], nc)

    zbias = jnp.zeros((1, 128), jnp.float32)

    gd0 = _score_cols(W0, [(0, as0), (1, as0), (0, ad0), (2, ad0)])
    gp0 = _score_cols(
        W0, [(2, as0), (3, as0), (5, as0), (1, ad0), (3, ad0), (4, ad0)])
    gc0 = _score_cols(W0, [(4, as0), (5, ad0)])
    yd = _proj_tc([hd0], zbias, jnp.stack([W0[0], W0[1], gd0]), nd, False)
    yp = _proj_tc([hp0], zbias,
                  jnp.stack([W0[2], W0[3], W0[5], gp0]), np_, False)
    yc = _proj_tc([hc0], zbias, jnp.stack([W0[4], gc0]), nc, False)

    sd = yd[2]
    sp = yp[3]
    sc = yc[1]

    def h2(y):
        return y.reshape(2 * y.shape[0], 64)

    od_dd = _gat_edge_sc(h2(yd[0]), sd[:, 0], sd[:, 2], s_dd, d_dd, nd, pad_d)
    op_dp = _gat_edge_sc(h2(yd[1]), sd[:, 1], sp[:, 3], s_dp, d_dp, np_, pad_p)
    od_rdp = _gat_edge_sc(
        h2(yp[0]), sp[:, 0], sd[:, 3], s_rdp, d_rdp, nd, pad_d)
    op_pp = _gat_edge_sc(h2(yp[1]), sp[:, 1], sp[:, 4], s_pp, d_pp, np_, pad_p)
    op_cp = _gat_edge_sc(h2(yc[0]), sc[:, 0], sp[:, 5], s_cp, d_cp, np_, pad_p)
    oc_rcp = _gat_edge_sc(
        h2(yp[2]), sp[:, 2], sc[:, 1], s_rcp, d_rcp, nc, pad_c)

    gd1 = _score_cols(W1, [(0, as1), (0, ad1), (2, ad1)])
    gp1 = _score_cols(W1, [(2, as1), (5, as1)])
    gc1 = _score_cols(W1, [(5, ad1)])
    bias_d = (b0[0] + b0[2]).reshape(1, 128)
    bias_p = (b0[1] + b0[3] + b0[4]).reshape(1, 128)
    bias_c = b0[5].reshape(1, 128)
    yd1 = _proj_tc([od_dd, od_rdp], bias_d, jnp.stack([W1[0], gd1]), nd, True)
    yp1 = _proj_tc([op_dp, op_pp, op_cp], bias_p,
                   jnp.stack([W1[2], W1[5], gp1]), np_, True)
    yc1 = _proj_tc([oc_rcp], bias_c, jnp.stack([gc1]), nc, True)

    sd1 = yd1[1]
    sp1 = yp1[2]
    sc1 = yc1[0]
    od_dd1 = _gat_edge_sc(
        h2(yd1[0]), sd1[:, 0], sd1[:, 1], s_dd, d_dd, nd, pad_d)
    od_rdp1 = _gat_edge_sc(
        h2(yp1[0]), sp1[:, 0], sd1[:, 2], s_rdp, d_rdp, nd, pad_d)
    oc_rcp1 = _gat_edge_sc(
        h2(yp1[1]), sp1[:, 1], sc1[:, 0], s_rcp, d_rcp, nc, pad_c)

    hd_fin = _finalize_tc(
        [od_dd1, od_rdp1], (b1[0] + b1[2]).reshape(1, 128), pad_d)
    hc_fin = _finalize_tc([oc_rcp1], b1[5].reshape(1, 128), pad_c)
    g1, g2, g3 = _readout_sc(hd_fin, hc_fin, drug1, drug2, cell)

    w3p = jnp.pad(cW3, ((0, 0), (0, 126)))
    b3p = jnp.pad(cb3, (0, 126)).reshape(1, 128)
    out = _mlp_tc(g1, g2, g3, cW1[:128], cW1[128:256], cW1[256:384],
                  cb1.reshape(1, 768), cW2, cb2.reshape(1, 256), w3p, b3p)
    return out[:, :2]


# speedup vs baseline: 11.4109x; 1.2890x over previous
import functools

import jax
import jax.numpy as jnp
from jax import lax
from jax.experimental import pallas as pl
from jax.experimental.pallas import tpu as pltpu
from jax.experimental.pallas import tpu_sc as plsc

_NS = 16
_K = 128


def _round_up(x, m):
    return ((x + m - 1) // m) * m


def _pick_bn(n, cap=2048):
    bn = 8
    for d in range(8, min(n, cap) + 1, 8):
        if n % d == 0:
            bn = d
    return bn


def _proj_tc(parts, bias_row, wst, n, relu):
    j_n = wst.shape[0]
    bn = _pick_bn(n)
    nb = n // bn
    npart = len(parts)

    def body(*refs):
        part_refs = refs[:npart]
        bias_ref = refs[npart]
        w_ref = refs[npart + 1]
        out_ref = refs[npart + 2]
        x = part_refs[0][...]
        for p in part_refs[1:]:
            x = x + p[...]
        x = x + bias_ref[...]
        if relu:
            x = jnp.maximum(x, 0.0)
        out_ref[0] = jnp.dot(x, w_ref[0], preferred_element_type=jnp.float32)

    in_specs = (
        [pl.BlockSpec((bn, 128), lambda i, j: (i, 0)) for _ in parts]
        + [pl.BlockSpec((1, 128), lambda i, j: (0, 0))]
        + [pl.BlockSpec((1, 128, 128), lambda i, j: (j, 0, 0))]
    )
    return pl.pallas_call(
        body,
        grid=(nb, j_n),
        in_specs=in_specs,
        out_specs=pl.BlockSpec((1, bn, 128), lambda i, j: (j, i, 0)),
        out_shape=jax.ShapeDtypeStruct((j_n, n, 128), jnp.float32),
    )(*parts, bias_row, wst)


def _finalize_tc(parts, bias_row, n_pad):
    bn = 1024 if n_pad % 1024 == 0 else 256
    npart = len(parts)

    def body(*refs):
        x = refs[0][...]
        for p in refs[1:npart]:
            x = x + p[...]
        x = x + refs[npart][...]
        refs[npart + 1][...] = jnp.maximum(x, 0.0)

    in_specs = [pl.BlockSpec((bn, 128), lambda i: (i, 0)) for _ in parts] + [
        pl.BlockSpec((1, 128), lambda i: (0, 0))
    ]
    return pl.pallas_call(
        body,
        grid=(n_pad // bn,),
        in_specs=in_specs,
        out_specs=pl.BlockSpec((bn, 128), lambda i: (i, 0)),
        out_shape=jax.ShapeDtypeStruct((n_pad, 128), jnp.float32),
    )(*parts, bias_row)


def _mlp_tc(g1, g2, g3, w1a, w1b, w1c, b1, w2, b2, w3, b3):
    b_n = g1.shape[0]
    bm = 512

    def body(g1r, g2r, g3r, w1ar, w1br, w1cr, b1r, w2r, b2r, w3r, b3r, outr):
        h = jnp.dot(g1r[...], w1ar[...], preferred_element_type=jnp.float32)
        h = h + jnp.dot(g2r[...], w1br[...], preferred_element_type=jnp.float32)
        h = h + jnp.dot(g3r[...], w1cr[...], preferred_element_type=jnp.float32)
        h = jnp.maximum(h + b1r[...], 0.0)
        h = jnp.maximum(
            jnp.dot(h, w2r[...], preferred_element_type=jnp.float32) + b2r[...],
            0.0,
        )
        outr[...] = (
            jnp.dot(h, w3r[...], preferred_element_type=jnp.float32) + b3r[...]
        )

    def full(shape):
        return pl.BlockSpec(shape, lambda i: tuple(0 for _ in shape))

    return pl.pallas_call(
        body,
        grid=(b_n // bm,),
        in_specs=[
            pl.BlockSpec((bm, 128), lambda i: (i, 0)),
            pl.BlockSpec((bm, 128), lambda i: (i, 0)),
            pl.BlockSpec((bm, 128), lambda i: (i, 0)),
            full((128, 768)),
            full((128, 768)),
            full((128, 768)),
            full((1, 768)),
            full((768, 256)),
            full((1, 256)),
            full((256, 128)),
            full((1, 128)),
        ],
        out_specs=pl.BlockSpec((bm, 128), lambda i: (i, 0)),
        out_shape=jax.ShapeDtypeStruct((b_n, 128), jnp.float32),
    )(g1, g2, g3, w1a, w1b, w1c, b1, w2, b2, w3, b3)


_POOL = 2097151


def _plan(n_src, n_dst):
    n_pad = _round_up(n_dst + 1, 256)
    free_tile = (_POOL - n_pad * 80) // _NS - 4096
    for k in (256, 128, 64):
        for ssrc in (True, False):
            for sdst in (True, False):
                use = k * (6 + 2 + 128 + 160) + 640 + 1280 + 1024
                use += (n_src if ssrc else 2 * k)
                use += (n_dst if sdst else 4 * k)
                if use <= free_tile:
                    return k, ssrc, sdst, n_pad
    return 64, False, False, n_pad


def _gat_edge_sc(h2, a_src, a_dst, src_e, dst_e, n_dst, n_pad, k_e, ssrc, sdst):
    n_src = a_src.shape[0]
    e_pad = src_e.shape[0]
    cw = e_pad // (_NS * k_e)
    r16 = n_pad // _NS
    cr = 16
    mesh = plsc.VectorSubcoreMesh(core_axis_name="c", subcore_axis_name="s")

    scratch = []
    for _ in range(2):
        scratch += [pltpu.VMEM((k_e,), jnp.int32)] * 3
        scratch += [pltpu.VMEM((k_e, 64), jnp.float32)]
        scratch += [pltpu.VMEM((k_e, 80), jnp.float32)]
        if not ssrc:
            scratch += [pltpu.VMEM((k_e,), jnp.float32)]
        if not sdst:
            scratch += [pltpu.VMEM((k_e,), jnp.int32)]
            scratch += [pltpu.VMEM((k_e,), jnp.float32)]
        scratch += [pltpu.SemaphoreType.DMA] * 2
    nper = 5 + (0 if ssrc else 1) + (0 if sdst else 2) + 2
    if ssrc:
        scratch += [pltpu.VMEM((n_src,), jnp.float32)]
    if sdst:
        scratch += [pltpu.VMEM((n_dst,), jnp.float32)]
    scratch += [
        pltpu.VMEM((k_e,), jnp.float32),
        pltpu.VMEM((8, 80), jnp.float32),
        pltpu.VMEM((cr, 80), jnp.float32),
        pltpu.VMEM((cr, 64), jnp.float32),
        pltpu.VMEM_SHARED((n_pad, 80), jnp.float32),
    ]

    @functools.partial(
        pl.kernel,
        out_type=jax.ShapeDtypeStruct((n_pad, 128), jnp.float32),
        mesh=mesh,
        compiler_params=pltpu.CompilerParams(
            use_tc_tiling_on_sc=False, needs_layout_passes=False),
        scratch_types=scratch,
    )
    def k(h2_hbm, asrc_hbm, adst_hbm, src_hbm, dst_hbm, out_hbm, *sc):
        bufs = {0: sc[:nper], 1: sc[nper:2 * nper]}
        rest = list(sc[2 * nper:])
        asrc_v = rest.pop(0) if ssrc else None
        adst_v = rest.pop(0) if sdst else None
        wbuf, zbuf, drainbuf, obuf, num_sh = rest
        c = lax.axis_index("c")
        s = lax.axis_index("s")
        zero16 = jnp.zeros((16,), jnp.float32)
        lanes = jnp.arange(16, dtype=jnp.int32)

        def parts(p):
            b = bufs[p]
            d = dict(sbuf=b[0], dbuf=b[1], gidx=b[2], rowbuf=b[3],
                     widebuf=b[4])
            i = 5
            if not ssrc:
                d["av"] = b[i]; i += 1
            if not sdst:
                d["dcl"] = b[i]; d["bv"] = b[i + 1]; i += 2
            d["sem_i"] = b[i]; d["sem_g"] = b[i + 1]
            return d

        def issue_idx(p, chunk):
            b = parts(p)
            base = chunk * k_e
            pltpu.async_copy(src_hbm.at[pl.ds(base, k_e)], b["sbuf"],
                             b["sem_i"])
            pltpu.async_copy(dst_hbm.at[pl.ds(base, k_e)], b["dbuf"],
                             b["sem_i"])

        def wait_idx(p, chunk):
            b = parts(p)
            base = chunk * k_e
            pltpu.make_async_copy(src_hbm.at[pl.ds(base, k_e)], b["sbuf"],
                                  b["sem_i"]).wait()
            pltpu.make_async_copy(dst_hbm.at[pl.ds(base, k_e)], b["dbuf"],
                                  b["sem_i"]).wait()

        def prep(p):
            b = parts(p)

            def pbody(j, carry):
                sl = pl.ds(j * 16, 16)
                b["gidx"][sl] = b["sbuf"][sl] * 2 + c
                if not sdst:
                    b["dcl"][sl] = jnp.minimum(b["dbuf"][sl], n_dst - 1)
                return carry

            lax.fori_loop(0, k_e // 16, pbody, 0)

        def issue_gat(p):
            b = parts(p)
            pltpu.async_copy(h2_hbm.at[b["gidx"]], b["rowbuf"], b["sem_g"])
            if not ssrc:
                pltpu.async_copy(asrc_hbm.at[b["sbuf"]], b["av"], b["sem_g"])
            if not sdst:
                pltpu.async_copy(adst_hbm.at[b["dcl"]], b["bv"], b["sem_g"])

        def wait_gat(p):
            b = parts(p)
            pltpu.make_async_copy(h2_hbm.at[b["gidx"]], b["rowbuf"],
                                  b["sem_g"]).wait()
            if not ssrc:
                pltpu.make_async_copy(asrc_hbm.at[b["sbuf"]], b["av"],
                                      b["sem_g"]).wait()
            if not sdst:
                pltpu.make_async_copy(adst_hbm.at[b["dcl"]], b["bv"],
                                      b["sem_g"]).wait()

        def scale_scatter(p):
            b = parts(p)

            def sbody(j, carry):
                sl = pl.ds(j * 16, 16)
                if ssrc:
                    aval = plsc.load_gather(asrc_v, [b["sbuf"][sl]])
                else:
                    aval = b["av"][sl]
                if sdst:
                    dv = jnp.minimum(b["dbuf"][sl], n_dst - 1)
                    bval = plsc.load_gather(adst_v, [dv])
                else:
                    bval = b["bv"][sl]
                al = aval + bval
                al = jnp.where(al >= 0.0, al, 0.2 * al)
                wbuf[sl] = jnp.exp(al)
                for l in range(16):
                    e = j * 16 + l
                    wb = plsc.load_gather(
                        wbuf, [jnp.full((16,), e, jnp.int32)])
                    for q in range(4):
                        ql = pl.ds(q * 16, 16)
                        b["widebuf"][e, ql] = b["rowbuf"][e, ql] * wb
                    b["widebuf"][e, pl.ds(64, 16)] = jnp.where(
                        lanes == 0, wb, 0.0)
                return carry

            lax.fori_loop(0, k_e // 16, sbody, 0)
            pltpu.sync_copy(b["widebuf"], num_sh.at[b["dbuf"]], add=True)

        for r in range(8):
            for q in range(5):
                zbuf[r, pl.ds(q * 16, 16)] = zero16

        def zbody(i, carry):
            pltpu.sync_copy(zbuf, num_sh.at[pl.ds(s * r16 + i * 8, 8)])
            return carry

        lax.fori_loop(0, r16 // 8, zbody, 0)
        if ssrc:
            pltpu.sync_copy(asrc_hbm, asrc_v)
        if sdst:
            pltpu.sync_copy(adst_hbm, adst_v)
        plsc.subcore_barrier()

        c0 = s * cw
        issue_idx(0, c0)
        wait_idx(0, c0)
        prep(0)
        issue_gat(0)

        def pair_body(i2, carry):
            a = c0 + 2 * i2
            nxt = jnp.minimum(a + 2, c0 + cw - 1)
            issue_idx(1, a + 1)
            wait_idx(1, a + 1)
            prep(1)
            issue_gat(1)
            wait_gat(0)
            scale_scatter(0)
            issue_idx(0, nxt)
            wait_idx(0, nxt)
            prep(0)
            issue_gat(0)
            wait_gat(1)
            scale_scatter(1)
            return carry

        lax.fori_loop(0, cw // 2, pair_body, 0)
        wait_gat(0)
        plsc.subcore_barrier()

        def drain_body(i, carry):
            r0 = s * r16 + i * cr
            pltpu.sync_copy(num_sh.at[pl.ds(r0, cr)], drainbuf)
            for r in range(cr):
                den = plsc.load_gather(
                    drainbuf,
                    [jnp.full((16,), r, jnp.int32),
                     jnp.full((16,), 64, jnp.int32)],
                )
                m = den > 0.0
                for q in range(4):
                    ql = pl.ds(q * 16, 16)
                    obuf[r, ql] = jnp.where(m, drainbuf[r, ql] / den, 0.0)
            pltpu.sync_copy(obuf, out_hbm.at[pl.ds(r0, cr), pl.ds(c * 64, 64)])
            return carry

        lax.fori_loop(0, r16 // cr, drain_body, 0)

    return k(h2, a_src, a_dst, src_e, dst_e)


def _readout_sc(hd, hc, drug1, drug2, cell):
    b_n = drug1.shape[0]
    rb = b_n // 32
    mesh = plsc.VectorSubcoreMesh(core_axis_name="c", subcore_axis_name="s")
    out_t = jax.ShapeDtypeStruct((b_n, 128), jnp.float32)

    @functools.partial(
        pl.kernel,
        out_type=(out_t, out_t, out_t),
        mesh=mesh,
        compiler_params=pltpu.CompilerParams(use_tc_tiling_on_sc=False, needs_layout_passes=False),
        scratch_types=[
            pltpu.VMEM((rb,), jnp.int32),
            pltpu.VMEM((rb, 128), jnp.float32),
            pltpu.SemaphoreType.DMA,
        ],
    )
    def k(hd_hbm, hc_hbm, d1_hbm, d2_hbm, cl_hbm, o1, o2, o3, idx_v, buf, sem):
        wid = lax.axis_index("s") * 2 + lax.axis_index("c")
        base = wid * rb
        for idx_hbm, tab_hbm, out_hbm in (
            (d1_hbm, hd_hbm, o1),
            (d2_hbm, hd_hbm, o2),
            (cl_hbm, hc_hbm, o3),
        ):
            pltpu.sync_copy(idx_hbm.at[pl.ds(base, rb)], idx_v)
            pltpu.async_copy(tab_hbm.at[idx_v], buf, sem).wait()
            pltpu.sync_copy(buf, out_hbm.at[pl.ds(base, rb)])

    return k(hd, hc, drug1, drug2, cell)


def _pad_edges(src, dst, n_dst, k_e):
    e = src.shape[0]
    e_pad = _round_up(e, _NS * k_e * 2)
    pad = e_pad - e
    src = jnp.concatenate([src, jnp.zeros((pad,), jnp.int32)])
    dst = jnp.concatenate([dst, jnp.full((pad,), n_dst, jnp.int32)])
    return src, dst


def _score_cols(w_l, specs):
    cols = [w_l[r] @ v[r] for (r, v) in specs]
    g = jnp.stack(cols, axis=1)
    return jnp.pad(g, ((0, 0), (0, 128 - g.shape[1])))


def _tail(plan):
    k_e, ssrc, sdst, n_pad = plan
    return (n_pad, k_e, ssrc, sdst)


def kernel(x_drug, x_protein, x_cell, edge_index_dd, edge_index_dp,
           edge_index_rev_dp, edge_index_pp, edge_index_cp, edge_index_rev_cp,
           drug1, drug2, cell, drug_table, protein_table, cell_table,
           W0, as0, ad0, b0, W1, as1, ad1, b1, cW1, cb1, cW2, cb2, cW3, cb3):
    nd = drug_table.shape[0]
    np_ = protein_table.shape[0]
    nc = cell_table.shape[0]
    pl_dd = _plan(nd, nd)
    pl_dp = _plan(nd, np_)
    pl_rdp = _plan(np_, nd)
    pl_pp = _plan(np_, np_)
    pl_cp = _plan(nc, np_)
    pl_rcp = _plan(np_, nc)
    pad_d = pl_dd[3]
    pad_p = pl_dp[3]
    pad_c = pl_rcp[3]

    hd0 = jnp.take(drug_table, x_drug, axis=0)
    hp0 = jnp.take(protein_table, x_protein, axis=0)
    hc0 = jnp.take(cell_table, x_cell, axis=0)

    ar_d = jnp.arange(nd, dtype=jnp.int32)
    ar_p = jnp.arange(np_, dtype=jnp.int32)
    s_dd, d_dd = _pad_edges(
        jnp.concatenate([edge_index_dd[0], ar_d]),
        jnp.concatenate([edge_index_dd[1], ar_d]), nd, pl_dd[0])
    s_dp, d_dp = _pad_edges(edge_index_dp[0], edge_index_dp[1], np_, pl_dp[0])
    s_rdp, d_rdp = _pad_edges(edge_index_rev_dp[0], edge_index_rev_dp[1], nd, pl_rdp[0])
    s_pp, d_pp = _pad_edges(
        jnp.concatenate([edge_index_pp[0], ar_p]),
        jnp.concatenate([edge_index_pp[1], ar_p]), np_, pl_pp[0])
    s_cp, d_cp = _pad_edges(edge_index_cp[0], edge_index_cp[1], np_, pl_cp[0])
    s_rcp, d_rcp = _pad_edges(edge_index_rev_cp[0], edge_index_rev_cp[1], nc, pl_rcp[0])

    zbias = jnp.zeros((1, 128), jnp.float32)

    gd0 = _score_cols(W0, [(0, as0), (1, as0), (0, ad0), (2, ad0)])
    gp0 = _score_cols(
        W0, [(2, as0), (3, as0), (5, as0), (1, ad0), (3, ad0), (4, ad0)])
    gc0 = _score_cols(W0, [(4, as0), (5, ad0)])
    yd = _proj_tc([hd0], zbias, jnp.stack([W0[0], W0[1], gd0]), nd, False)
    yp = _proj_tc([hp0], zbias,
                  jnp.stack([W0[2], W0[3], W0[5], gp0]), np_, False)
    yc = _proj_tc([hc0], zbias, jnp.stack([W0[4], gc0]), nc, False)

    sd = yd[2]
    sp = yp[3]
    sc = yc[1]

    def h2(y):
        return y.reshape(2 * y.shape[0], 64)

    od_dd = _gat_edge_sc(h2(yd[0]), sd[:, 0], sd[:, 2], s_dd, d_dd, nd, *_tail(pl_dd))
    op_dp = _gat_edge_sc(h2(yd[1]), sd[:, 1], sp[:, 3], s_dp, d_dp, np_, *_tail(pl_dp))
    od_rdp = _gat_edge_sc(
        h2(yp[0]), sp[:, 0], sd[:, 3], s_rdp, d_rdp, nd, *_tail(pl_rdp))
    op_pp = _gat_edge_sc(h2(yp[1]), sp[:, 1], sp[:, 4], s_pp, d_pp, np_, *_tail(pl_pp))
    op_cp = _gat_edge_sc(h2(yc[0]), sc[:, 0], sp[:, 5], s_cp, d_cp, np_, *_tail(pl_cp))
    oc_rcp = _gat_edge_sc(
        h2(yp[2]), sp[:, 2], sc[:, 1], s_rcp, d_rcp, nc, *_tail(pl_rcp))

    gd1 = _score_cols(W1, [(0, as1), (0, ad1), (2, ad1)])
    gp1 = _score_cols(W1, [(2, as1), (5, as1)])
    gc1 = _score_cols(W1, [(5, ad1)])
    bias_d = (b0[0] + b0[2]).reshape(1, 128)
    bias_p = (b0[1] + b0[3] + b0[4]).reshape(1, 128)
    bias_c = b0[5].reshape(1, 128)
    yd1 = _proj_tc([od_dd, od_rdp], bias_d, jnp.stack([W1[0], gd1]), nd, True)
    yp1 = _proj_tc([op_dp, op_pp, op_cp], bias_p,
                   jnp.stack([W1[2], W1[5], gp1]), np_, True)
    yc1 = _proj_tc([oc_rcp], bias_c, jnp.stack([gc1]), nc, True)

    sd1 = yd1[1]
    sp1 = yp1[2]
    sc1 = yc1[0]
    od_dd1 = _gat_edge_sc(
        h2(yd1[0]), sd1[:, 0], sd1[:, 1], s_dd, d_dd, nd, *_tail(pl_dd))
    od_rdp1 = _gat_edge_sc(
        h2(yp1[0]), sp1[:, 0], sd1[:, 2], s_rdp, d_rdp, nd, *_tail(pl_rdp))
    oc_rcp1 = _gat_edge_sc(
        h2(yp1[1]), sp1[:, 1], sc1[:, 0], s_rcp, d_rcp, nc, *_tail(pl_rcp))

    hd_fin = _finalize_tc(
        [od_dd1, od_rdp1], (b1[0] + b1[2]).reshape(1, 128), pad_d)
    hc_fin = _finalize_tc([oc_rcp1], b1[5].reshape(1, 128), pad_c)
    g1, g2, g3 = _readout_sc(hd_fin, hc_fin, drug1, drug2, cell)

    w3p = jnp.pad(cW3, ((0, 0), (0, 126)))
    b3p = jnp.pad(cb3, (0, 126)).reshape(1, 128)
    out = _mlp_tc(g1, g2, g3, cW1[:128], cW1[128:256], cW1[256:384],
                  cb1.reshape(1, 768), cW2, cb2.reshape(1, 256), w3p, b3p)
    return out[:, :2]


# speedup vs baseline: 12.5445x; 1.0993x over previous
import functools

import jax
import jax.numpy as jnp
from jax import lax
from jax.experimental import pallas as pl
from jax.experimental.pallas import tpu as pltpu
from jax.experimental.pallas import tpu_sc as plsc

_NS = 16
_K = 128


def _round_up(x, m):
    return ((x + m - 1) // m) * m


def _pick_bn(n, cap=2048):
    bn = 8
    for d in range(8, min(n, cap) + 1, 8):
        if n % d == 0:
            bn = d
    return bn


def _proj_tc(parts, bias_row, wst, n, relu):
    j_n = wst.shape[0]
    bn = _pick_bn(n)
    nb = n // bn
    npart = len(parts)

    def body(*refs):
        part_refs = refs[:npart]
        bias_ref = refs[npart]
        w_ref = refs[npart + 1]
        out_ref = refs[npart + 2]
        x = part_refs[0][...]
        for p in part_refs[1:]:
            x = x + p[...]
        x = x + bias_ref[...]
        if relu:
            x = jnp.maximum(x, 0.0)
        out_ref[0] = jnp.dot(x, w_ref[0], preferred_element_type=jnp.float32)

    in_specs = (
        [pl.BlockSpec((bn, 128), lambda i, j: (i, 0)) for _ in parts]
        + [pl.BlockSpec((1, 128), lambda i, j: (0, 0))]
        + [pl.BlockSpec((1, 128, 128), lambda i, j: (j, 0, 0))]
    )
    return pl.pallas_call(
        body,
        grid=(nb, j_n),
        in_specs=in_specs,
        out_specs=pl.BlockSpec((1, bn, 128), lambda i, j: (j, i, 0)),
        out_shape=jax.ShapeDtypeStruct((j_n, n, 128), jnp.float32),
    )(*parts, bias_row, wst)


def _finalize_tc(parts, bias_row, n_pad):
    bn = 1024 if n_pad % 1024 == 0 else 256
    npart = len(parts)

    def body(*refs):
        x = refs[0][...]
        for p in refs[1:npart]:
            x = x + p[...]
        x = x + refs[npart][...]
        refs[npart + 1][...] = jnp.maximum(x, 0.0)

    in_specs = [pl.BlockSpec((bn, 128), lambda i: (i, 0)) for _ in parts] + [
        pl.BlockSpec((1, 128), lambda i: (0, 0))
    ]
    return pl.pallas_call(
        body,
        grid=(n_pad // bn,),
        in_specs=in_specs,
        out_specs=pl.BlockSpec((bn, 128), lambda i: (i, 0)),
        out_shape=jax.ShapeDtypeStruct((n_pad, 128), jnp.float32),
    )(*parts, bias_row)


def _mlp_tc(g1, g2, g3, w1a, w1b, w1c, b1, w2, b2, w3, b3):
    b_n = g1.shape[0]
    bm = 512

    def body(g1r, g2r, g3r, w1ar, w1br, w1cr, b1r, w2r, b2r, w3r, b3r, outr):
        h = jnp.dot(g1r[...], w1ar[...], preferred_element_type=jnp.float32)
        h = h + jnp.dot(g2r[...], w1br[...], preferred_element_type=jnp.float32)
        h = h + jnp.dot(g3r[...], w1cr[...], preferred_element_type=jnp.float32)
        h = jnp.maximum(h + b1r[...], 0.0)
        h = jnp.maximum(
            jnp.dot(h, w2r[...], preferred_element_type=jnp.float32) + b2r[...],
            0.0,
        )
        outr[...] = (
            jnp.dot(h, w3r[...], preferred_element_type=jnp.float32) + b3r[...]
        )

    def full(shape):
        return pl.BlockSpec(shape, lambda i: tuple(0 for _ in shape))

    return pl.pallas_call(
        body,
        grid=(b_n // bm,),
        in_specs=[
            pl.BlockSpec((bm, 128), lambda i: (i, 0)),
            pl.BlockSpec((bm, 128), lambda i: (i, 0)),
            pl.BlockSpec((bm, 128), lambda i: (i, 0)),
            full((128, 768)),
            full((128, 768)),
            full((128, 768)),
            full((1, 768)),
            full((768, 256)),
            full((1, 256)),
            full((256, 128)),
            full((1, 128)),
        ],
        out_specs=pl.BlockSpec((bm, 128), lambda i: (i, 0)),
        out_shape=jax.ShapeDtypeStruct((b_n, 128), jnp.float32),
    )(g1, g2, g3, w1a, w1b, w1c, b1, w2, b2, w3, b3)


_POOL = 2097151


def _plan(n_src, n_dst):
    n_pad = _round_up(n_dst + 1, 512)
    free_tile = (_POOL - n_pad * 80) // _NS - 4096
    for k in (256, 128, 64):
        for ssrc in (True, False):
            for sdst in (True, False):
                use = k * (6 + 2 + 128 + 160) + 640 + 1280 + 1024
                use += (n_src if ssrc else 2 * k)
                use += (n_dst if sdst else 4 * k)
                if use <= free_tile:
                    return k, ssrc, sdst, n_pad
    return 64, False, False, n_pad


def _gat_edge_sc(h2, a_src, a_dst, src_e, dst_e, n_dst, n_pad, k_e, ssrc, sdst):
    n_src = a_src.shape[0]
    e_pad = src_e.shape[0]
    cw = e_pad // (_NS * k_e)
    r16 = n_pad // _NS
    cr = 32
    mesh = plsc.VectorSubcoreMesh(core_axis_name="c", subcore_axis_name="s")

    scratch = []
    for _ in range(2):
        scratch += [pltpu.VMEM((k_e,), jnp.int32)] * 3
        scratch += [pltpu.VMEM((k_e, 64), jnp.float32)]
        scratch += [pltpu.VMEM((k_e, 80), jnp.float32)]
        if not ssrc:
            scratch += [pltpu.VMEM((k_e,), jnp.float32)]
        if not sdst:
            scratch += [pltpu.VMEM((k_e,), jnp.int32)]
            scratch += [pltpu.VMEM((k_e,), jnp.float32)]
        scratch += [pltpu.VMEM((k_e,), jnp.int32)]
        scratch += [pltpu.SemaphoreType.DMA] * 3
    nper = 5 + (0 if ssrc else 1) + (0 if sdst else 2) + 4
    if ssrc:
        scratch += [pltpu.VMEM((n_src,), jnp.float32)]
    if sdst:
        scratch += [pltpu.VMEM((n_dst,), jnp.float32)]
    scratch += [
        pltpu.VMEM((k_e,), jnp.float32),
        pltpu.VMEM((cr, 80), jnp.float32),
        pltpu.VMEM((cr, 64), jnp.float32),
        pltpu.VMEM_SHARED((n_pad, 80), jnp.float32),
    ]

    @functools.partial(
        pl.kernel,
        out_type=jax.ShapeDtypeStruct((n_pad, 128), jnp.float32),
        mesh=mesh,
        compiler_params=pltpu.CompilerParams(
            use_tc_tiling_on_sc=False, needs_layout_passes=False),
        scratch_types=scratch,
    )
    def k(h2_hbm, asrc_hbm, adst_hbm, src_hbm, dst_hbm, out_hbm, *sc):
        bufs = {0: sc[:nper], 1: sc[nper:2 * nper]}
        rest = list(sc[2 * nper:])
        asrc_v = rest.pop(0) if ssrc else None
        adst_v = rest.pop(0) if sdst else None
        wbuf, drainbuf, obuf, num_sh = rest
        c = lax.axis_index("c")
        s = lax.axis_index("s")
        zero16 = jnp.zeros((16,), jnp.float32)
        lanes = jnp.arange(16, dtype=jnp.int32)

        def parts(p):
            b = bufs[p]
            d = dict(sbuf=b[0], dbuf=b[1], gidx=b[2], rowbuf=b[3],
                     widebuf=b[4])
            i = 5
            if not ssrc:
                d["av"] = b[i]; i += 1
            if not sdst:
                d["dcl"] = b[i]; d["bv"] = b[i + 1]; i += 2
            d["dscat"] = b[i]; i += 1
            d["sem_i"] = b[i]; d["sem_g"] = b[i + 1]; d["sem_s"] = b[i + 2]
            return d

        def issue_idx(p, chunk):
            b = parts(p)
            base = chunk * k_e
            pltpu.async_copy(src_hbm.at[pl.ds(base, k_e)], b["sbuf"],
                             b["sem_i"])
            pltpu.async_copy(dst_hbm.at[pl.ds(base, k_e)], b["dbuf"],
                             b["sem_i"])

        def wait_idx(p, chunk):
            b = parts(p)
            base = chunk * k_e
            pltpu.make_async_copy(src_hbm.at[pl.ds(base, k_e)], b["sbuf"],
                                  b["sem_i"]).wait()
            pltpu.make_async_copy(dst_hbm.at[pl.ds(base, k_e)], b["dbuf"],
                                  b["sem_i"]).wait()

        def prep(p):
            b = parts(p)

            def pbody(j, carry):
                sl = pl.ds(j * 16, 16)
                b["gidx"][sl] = b["sbuf"][sl] * 2 + c
                if not sdst:
                    b["dcl"][sl] = jnp.minimum(b["dbuf"][sl], n_dst - 1)
                return carry

            lax.fori_loop(0, k_e // 16, pbody, 0)

        def issue_gat(p):
            b = parts(p)
            pltpu.async_copy(h2_hbm.at[b["gidx"]], b["rowbuf"], b["sem_g"])
            if not ssrc:
                pltpu.async_copy(asrc_hbm.at[b["sbuf"]], b["av"], b["sem_g"])
            if not sdst:
                pltpu.async_copy(adst_hbm.at[b["dcl"]], b["bv"], b["sem_g"])

        def wait_gat(p):
            b = parts(p)
            pltpu.make_async_copy(h2_hbm.at[b["gidx"]], b["rowbuf"],
                                  b["sem_g"]).wait()
            if not ssrc:
                pltpu.make_async_copy(asrc_hbm.at[b["sbuf"]], b["av"],
                                      b["sem_g"]).wait()
            if not sdst:
                pltpu.make_async_copy(adst_hbm.at[b["dcl"]], b["bv"],
                                      b["sem_g"]).wait()

        def wait_scat(p):
            b = parts(p)
            pltpu.make_async_copy(b["widebuf"], num_sh.at[b["dscat"]],
                                  b["sem_s"]).wait()

        def scale_scatter(p):
            b = parts(p)
            wait_scat(p)

            def sbody(j, carry):
                sl = pl.ds(j * 16, 16)
                if ssrc:
                    aval = plsc.load_gather(asrc_v, [b["sbuf"][sl]])
                else:
                    aval = b["av"][sl]
                if sdst:
                    dv = jnp.minimum(b["dbuf"][sl], n_dst - 1)
                    bval = plsc.load_gather(adst_v, [dv])
                else:
                    bval = b["bv"][sl]
                al = aval + bval
                al = jnp.where(al >= 0.0, al, 0.2 * al)
                wbuf[sl] = jnp.exp(al)
                for l in range(16):
                    e = j * 16 + l
                    wb = plsc.load_gather(
                        wbuf, [jnp.full((16,), e, jnp.int32)])
                    for q in range(4):
                        ql = pl.ds(q * 16, 16)
                        b["widebuf"][e, ql] = b["rowbuf"][e, ql] * wb
                    b["widebuf"][e, pl.ds(64, 16)] = jnp.where(
                        lanes == 0, wb, 0.0)
                b["dscat"][sl] = b["dbuf"][sl]
                return carry

            lax.fori_loop(0, k_e // 16, sbody, 0)
            pltpu.async_copy(b["widebuf"], num_sh.at[b["dscat"]], b["sem_s"],
                             add=True)

        jrow = jnp.full((16,), n_dst, jnp.int32)

        def wzero(p):
            b = parts(p)

            def wzbody(r, carry):
                for q in range(5):
                    b["widebuf"][r, pl.ds(q * 16, 16)] = zero16
                return carry

            lax.fori_loop(0, k_e, wzbody, 0)

            def dzbody(j, carry):
                b["dscat"][pl.ds(j * 16, 16)] = jrow
                return carry

            lax.fori_loop(0, k_e // 16, dzbody, 0)

        wzero(0)
        wzero(1)
        zc = min(k_e, r16)
        while r16 % zc:
            zc //= 2
        wb0 = parts(0)["widebuf"]

        def zbody(i, carry):
            pltpu.sync_copy(wb0.at[pl.ds(0, zc)],
                            num_sh.at[pl.ds(s * r16 + i * zc, zc)])
            return carry

        lax.fori_loop(0, r16 // zc, zbody, 0)
        pltpu.async_copy(parts(0)["widebuf"], num_sh.at[parts(0)["dscat"]],
                         parts(0)["sem_s"], add=True)
        pltpu.async_copy(parts(1)["widebuf"], num_sh.at[parts(1)["dscat"]],
                         parts(1)["sem_s"], add=True)
        if ssrc:
            pltpu.sync_copy(asrc_hbm, asrc_v)
        if sdst:
            pltpu.sync_copy(adst_hbm, adst_v)
        plsc.subcore_barrier()

        c0 = s * cw
        issue_idx(0, c0)
        wait_idx(0, c0)
        prep(0)
        issue_gat(0)

        def pair_body(i2, carry):
            a = c0 + 2 * i2
            nxt = jnp.minimum(a + 2, c0 + cw - 1)
            issue_idx(1, a + 1)
            wait_idx(1, a + 1)
            prep(1)
            issue_gat(1)
            wait_gat(0)
            scale_scatter(0)
            issue_idx(0, nxt)
            wait_idx(0, nxt)
            prep(0)
            issue_gat(0)
            wait_gat(1)
            scale_scatter(1)
            return carry

        lax.fori_loop(0, cw // 2, pair_body, 0)
        wait_gat(0)
        wait_scat(0)
        wait_scat(1)
        plsc.subcore_barrier()

        def drain_body(i, carry):
            r0 = s * r16 + i * cr
            pltpu.sync_copy(num_sh.at[pl.ds(r0, cr)], drainbuf)
            for r in range(cr):
                den = plsc.load_gather(
                    drainbuf,
                    [jnp.full((16,), r, jnp.int32),
                     jnp.full((16,), 64, jnp.int32)],
                )
                m = den > 0.0
                for q in range(4):
                    ql = pl.ds(q * 16, 16)
                    obuf[r, ql] = jnp.where(m, drainbuf[r, ql] / den, 0.0)
            pltpu.sync_copy(obuf, out_hbm.at[pl.ds(r0, cr), pl.ds(c * 64, 64)])
            return carry

        lax.fori_loop(0, r16 // cr, drain_body, 0)

    return k(h2, a_src, a_dst, src_e, dst_e)


def _readout_sc(hd, hc, drug1, drug2, cell):
    b_n = drug1.shape[0]
    rb = b_n // 32
    mesh = plsc.VectorSubcoreMesh(core_axis_name="c", subcore_axis_name="s")
    out_t = jax.ShapeDtypeStruct((b_n, 128), jnp.float32)

    @functools.partial(
        pl.kernel,
        out_type=(out_t, out_t, out_t),
        mesh=mesh,
        compiler_params=pltpu.CompilerParams(use_tc_tiling_on_sc=False, needs_layout_passes=False),
        scratch_types=[
            pltpu.VMEM((rb,), jnp.int32),
            pltpu.VMEM((rb, 128), jnp.float32),
            pltpu.SemaphoreType.DMA,
        ],
    )
    def k(hd_hbm, hc_hbm, d1_hbm, d2_hbm, cl_hbm, o1, o2, o3, idx_v, buf, sem):
        wid = lax.axis_index("s") * 2 + lax.axis_index("c")
        base = wid * rb
        for idx_hbm, tab_hbm, out_hbm in (
            (d1_hbm, hd_hbm, o1),
            (d2_hbm, hd_hbm, o2),
            (cl_hbm, hc_hbm, o3),
        ):
            pltpu.sync_copy(idx_hbm.at[pl.ds(base, rb)], idx_v)
            pltpu.async_copy(tab_hbm.at[idx_v], buf, sem).wait()
            pltpu.sync_copy(buf, out_hbm.at[pl.ds(base, rb)])

    return k(hd, hc, drug1, drug2, cell)


def _pad_edges(src, dst, n_dst, k_e):
    e = src.shape[0]
    e_pad = _round_up(e, _NS * k_e * 2)
    pad = e_pad - e
    src = jnp.concatenate([src, jnp.zeros((pad,), jnp.int32)])
    dst = jnp.concatenate([dst, jnp.full((pad,), n_dst, jnp.int32)])
    return src, dst


def _score_cols(w_l, specs):
    cols = [w_l[r] @ v[r] for (r, v) in specs]
    g = jnp.stack(cols, axis=1)
    return jnp.pad(g, ((0, 0), (0, 128 - g.shape[1])))


def _tail(plan):
    k_e, ssrc, sdst, n_pad = plan
    return (n_pad, k_e, ssrc, sdst)


def kernel(x_drug, x_protein, x_cell, edge_index_dd, edge_index_dp,
           edge_index_rev_dp, edge_index_pp, edge_index_cp, edge_index_rev_cp,
           drug1, drug2, cell, drug_table, protein_table, cell_table,
           W0, as0, ad0, b0, W1, as1, ad1, b1, cW1, cb1, cW2, cb2, cW3, cb3):
    nd = drug_table.shape[0]
    np_ = protein_table.shape[0]
    nc = cell_table.shape[0]
    pl_dd = _plan(nd, nd)
    pl_dp = _plan(nd, np_)
    pl_rdp = _plan(np_, nd)
    pl_pp = _plan(np_, np_)
    pl_cp = _plan(nc, np_)
    pl_rcp = _plan(np_, nc)
    pad_d = pl_dd[3]
    pad_p = pl_dp[3]
    pad_c = pl_rcp[3]

    hd0 = jnp.take(drug_table, x_drug, axis=0)
    hp0 = jnp.take(protein_table, x_protein, axis=0)
    hc0 = jnp.take(cell_table, x_cell, axis=0)

    ar_d = jnp.arange(nd, dtype=jnp.int32)
    ar_p = jnp.arange(np_, dtype=jnp.int32)
    s_dd, d_dd = _pad_edges(
        jnp.concatenate([edge_index_dd[0], ar_d]),
        jnp.concatenate([edge_index_dd[1], ar_d]), nd, pl_dd[0])
    s_dp, d_dp = _pad_edges(edge_index_dp[0], edge_index_dp[1], np_, pl_dp[0])
    s_rdp, d_rdp = _pad_edges(edge_index_rev_dp[0], edge_index_rev_dp[1], nd, pl_rdp[0])
    s_pp, d_pp = _pad_edges(
        jnp.concatenate([edge_index_pp[0], ar_p]),
        jnp.concatenate([edge_index_pp[1], ar_p]), np_, pl_pp[0])
    s_cp, d_cp = _pad_edges(edge_index_cp[0], edge_index_cp[1], np_, pl_cp[0])
    s_rcp, d_rcp = _pad_edges(edge_index_rev_cp[0], edge_index_rev_cp[1], nc, pl_rcp[0])

    zbias = jnp.zeros((1, 128), jnp.float32)

    gd0 = _score_cols(W0, [(0, as0), (1, as0), (0, ad0), (2, ad0)])
    gp0 = _score_cols(
        W0, [(2, as0), (3, as0), (5, as0), (1, ad0), (3, ad0), (4, ad0)])
    gc0 = _score_cols(W0, [(4, as0), (5, ad0)])
    yd = _proj_tc([hd0], zbias, jnp.stack([W0[0], W0[1], gd0]), nd, False)
    yp = _proj_tc([hp0], zbias,
                  jnp.stack([W0[2], W0[3], W0[5], gp0]), np_, False)
    yc = _proj_tc([hc0], zbias, jnp.stack([W0[4], gc0]), nc, False)

    sd = yd[2]
    sp = yp[3]
    sc = yc[1]

    def h2(y):
        return y.reshape(2 * y.shape[0], 64)

    od_dd = _gat_edge_sc(h2(yd[0]), sd[:, 0], sd[:, 2], s_dd, d_dd, nd, *_tail(pl_dd))
    op_dp = _gat_edge_sc(h2(yd[1]), sd[:, 1], sp[:, 3], s_dp, d_dp, np_, *_tail(pl_dp))
    od_rdp = _gat_edge_sc(
        h2(yp[0]), sp[:, 0], sd[:, 3], s_rdp, d_rdp, nd, *_tail(pl_rdp))
    op_pp = _gat_edge_sc(h2(yp[1]), sp[:, 1], sp[:, 4], s_pp, d_pp, np_, *_tail(pl_pp))
    op_cp = _gat_edge_sc(h2(yc[0]), sc[:, 0], sp[:, 5], s_cp, d_cp, np_, *_tail(pl_cp))
    oc_rcp = _gat_edge_sc(
        h2(yp[2]), sp[:, 2], sc[:, 1], s_rcp, d_rcp, nc, *_tail(pl_rcp))

    gd1 = _score_cols(W1, [(0, as1), (0, ad1), (2, ad1)])
    gp1 = _score_cols(W1, [(2, as1), (5, as1)])
    gc1 = _score_cols(W1, [(5, ad1)])
    bias_d = (b0[0] + b0[2]).reshape(1, 128)
    bias_p = (b0[1] + b0[3] + b0[4]).reshape(1, 128)
    bias_c = b0[5].reshape(1, 128)
    yd1 = _proj_tc([od_dd, od_rdp], bias_d, jnp.stack([W1[0], gd1]), nd, True)
    yp1 = _proj_tc([op_dp, op_pp, op_cp], bias_p,
                   jnp.stack([W1[2], W1[5], gp1]), np_, True)
    yc1 = _proj_tc([oc_rcp], bias_c, jnp.stack([gc1]), nc, True)

    sd1 = yd1[1]
    sp1 = yp1[2]
    sc1 = yc1[0]
    od_dd1 = _gat_edge_sc(
        h2(yd1[0]), sd1[:, 0], sd1[:, 1], s_dd, d_dd, nd, *_tail(pl_dd))
    od_rdp1 = _gat_edge_sc(
        h2(yp1[0]), sp1[:, 0], sd1[:, 2], s_rdp, d_rdp, nd, *_tail(pl_rdp))
    oc_rcp1 = _gat_edge_sc(
        h2(yp1[1]), sp1[:, 1], sc1[:, 0], s_rcp, d_rcp, nc, *_tail(pl_rcp))

    hd_fin = _finalize_tc(
        [od_dd1, od_rdp1], (b1[0] + b1[2]).reshape(1, 128), pad_d)
    hc_fin = _finalize_tc([oc_rcp1], b1[5].reshape(1, 128), pad_c)
    g1, g2, g3 = _readout_sc(hd_fin, hc_fin, drug1, drug2, cell)

    w3p = jnp.pad(cW3, ((0, 0), (0, 126)))
    b3p = jnp.pad(cb3, (0, 126)).reshape(1, 128)
    out = _mlp_tc(g1, g2, g3, cW1[:128], cW1[128:256], cW1[256:384],
                  cb1.reshape(1, 768), cW2, cb2.reshape(1, 256), w3p, b3p)
    return out[:, :2]


# speedup vs baseline: 15.6565x; 1.2481x over previous
import functools

import jax
import jax.numpy as jnp
from jax import lax
from jax.experimental import pallas as pl
from jax.experimental.pallas import tpu as pltpu
from jax.experimental.pallas import tpu_sc as plsc

_NS = 16
_K = 128


def _round_up(x, m):
    return ((x + m - 1) // m) * m


def _pick_bn(n, cap=2048):
    bn = 8
    for d in range(8, min(n, cap) + 1, 8):
        if n % d == 0:
            bn = d
    return bn


def _proj_tc(parts, bias_row, wst, n, relu):
    j_n = wst.shape[0]
    bn = _pick_bn(n)
    nb = n // bn
    npart = len(parts)

    def body(*refs):
        part_refs = refs[:npart]
        bias_ref = refs[npart]
        w_ref = refs[npart + 1]
        out_ref = refs[npart + 2]
        x = part_refs[0][...]
        for p in part_refs[1:]:
            x = x + p[...]
        x = x + bias_ref[...]
        if relu:
            x = jnp.maximum(x, 0.0)
        out_ref[0] = jnp.dot(x, w_ref[0], preferred_element_type=jnp.float32)

    in_specs = (
        [pl.BlockSpec((bn, 128), lambda i, j: (i, 0)) for _ in parts]
        + [pl.BlockSpec((1, 128), lambda i, j: (0, 0))]
        + [pl.BlockSpec((1, 128, 128), lambda i, j: (j, 0, 0))]
    )
    return pl.pallas_call(
        body,
        grid=(nb, j_n),
        in_specs=in_specs,
        out_specs=pl.BlockSpec((1, bn, 128), lambda i, j: (j, i, 0)),
        out_shape=jax.ShapeDtypeStruct((j_n, n, 128), jnp.float32),
    )(*parts, bias_row, wst)


def _finalize_tc(parts, bias_row, n_pad):
    bn = 1024 if n_pad % 1024 == 0 else 256
    npart = len(parts)

    def body(*refs):
        x = refs[0][...]
        for p in refs[1:npart]:
            x = x + p[...]
        x = x + refs[npart][...]
        refs[npart + 1][...] = jnp.maximum(x, 0.0)

    in_specs = [pl.BlockSpec((bn, 128), lambda i: (i, 0)) for _ in parts] + [
        pl.BlockSpec((1, 128), lambda i: (0, 0))
    ]
    return pl.pallas_call(
        body,
        grid=(n_pad // bn,),
        in_specs=in_specs,
        out_specs=pl.BlockSpec((bn, 128), lambda i: (i, 0)),
        out_shape=jax.ShapeDtypeStruct((n_pad, 128), jnp.float32),
    )(*parts, bias_row)


def _mlp_tc(g1, g2, g3, w1a, w1b, w1c, b1, w2, b2, w3, b3):
    b_n = g1.shape[0]
    bm = 512

    def body(g1r, g2r, g3r, w1ar, w1br, w1cr, b1r, w2r, b2r, w3r, b3r, outr):
        h = jnp.dot(g1r[...], w1ar[...], preferred_element_type=jnp.float32)
        h = h + jnp.dot(g2r[...], w1br[...], preferred_element_type=jnp.float32)
        h = h + jnp.dot(g3r[...], w1cr[...], preferred_element_type=jnp.float32)
        h = jnp.maximum(h + b1r[...], 0.0)
        h = jnp.maximum(
            jnp.dot(h, w2r[...], preferred_element_type=jnp.float32) + b2r[...],
            0.0,
        )
        outr[...] = (
            jnp.dot(h, w3r[...], preferred_element_type=jnp.float32) + b3r[...]
        )

    def full(shape):
        return pl.BlockSpec(shape, lambda i: tuple(0 for _ in shape))

    return pl.pallas_call(
        body,
        grid=(b_n // bm,),
        in_specs=[
            pl.BlockSpec((bm, 128), lambda i: (i, 0)),
            pl.BlockSpec((bm, 128), lambda i: (i, 0)),
            pl.BlockSpec((bm, 128), lambda i: (i, 0)),
            full((128, 768)),
            full((128, 768)),
            full((128, 768)),
            full((1, 768)),
            full((768, 256)),
            full((1, 256)),
            full((256, 128)),
            full((1, 128)),
        ],
        out_specs=pl.BlockSpec((bm, 128), lambda i: (i, 0)),
        out_shape=jax.ShapeDtypeStruct((b_n, 128), jnp.float32),
    )(g1, g2, g3, w1a, w1b, w1c, b1, w2, b2, w3, b3)


_POOL = 2097151


def _plan(n_src, n_dst):
    n_pad = _round_up(n_dst + 1, 512)
    free_tile = (_POOL - n_pad * 80) // _NS - 4096
    for k in (256, 128, 64):
        for ssrc in (True, False):
            for sdst in (True, False):
                use = k * (6 + 2 + 128 + 160) + 640 + 1280 + 1024
                use += (n_src if ssrc else 2 * k)
                use += (n_dst if sdst else 4 * k)
                if use <= free_tile:
                    return k, ssrc, sdst, n_pad
    return 64, False, False, n_pad


def _gat_edge_sc(h2, a_src, a_dst, src_e, dst_e, n_dst, n_pad, k_e, ssrc, sdst):
    n_src = a_src.shape[0]
    e_pad = src_e.shape[0]
    cw = e_pad // (_NS * k_e)
    r16 = n_pad // _NS
    cr = 32
    mesh = plsc.VectorSubcoreMesh(core_axis_name="c", subcore_axis_name="s")

    scratch = []
    for _ in range(2):
        scratch += [pltpu.VMEM((k_e,), jnp.int32)] * 3
        scratch += [pltpu.VMEM((k_e, 64), jnp.float32)]
        scratch += [pltpu.VMEM((k_e, 80), jnp.float32)]
        if not ssrc:
            scratch += [pltpu.VMEM((k_e,), jnp.float32)]
        if not sdst:
            scratch += [pltpu.VMEM((k_e,), jnp.int32)]
            scratch += [pltpu.VMEM((k_e,), jnp.float32)]
        scratch += [pltpu.VMEM((k_e,), jnp.int32)]
        scratch += [pltpu.SemaphoreType.DMA] * 3
    nper = 5 + (0 if ssrc else 1) + (0 if sdst else 2) + 4
    if ssrc:
        scratch += [pltpu.VMEM((n_src,), jnp.float32)]
    if sdst:
        scratch += [pltpu.VMEM((n_dst,), jnp.float32)]
    scratch += [
        pltpu.VMEM((k_e,), jnp.float32),
        pltpu.VMEM((cr, 80), jnp.float32),
        pltpu.VMEM((cr, 64), jnp.float32),
        pltpu.VMEM_SHARED((n_pad, 80), jnp.float32),
    ]

    @functools.partial(
        pl.kernel,
        out_type=jax.ShapeDtypeStruct((n_pad, 128), jnp.float32),
        mesh=mesh,
        compiler_params=pltpu.CompilerParams(
            use_tc_tiling_on_sc=False, needs_layout_passes=False),
        scratch_types=scratch,
    )
    def k(h2_hbm, asrc_hbm, adst_hbm, src_hbm, dst_hbm, out_hbm, *sc):
        bufs = {0: sc[:nper], 1: sc[nper:2 * nper]}
        rest = list(sc[2 * nper:])
        asrc_v = rest.pop(0) if ssrc else None
        adst_v = rest.pop(0) if sdst else None
        wbuf, drainbuf, obuf, num_sh = rest
        c = lax.axis_index("c")
        s = lax.axis_index("s")
        zero16 = jnp.zeros((16,), jnp.float32)
        lanes = jnp.arange(16, dtype=jnp.int32)

        def parts(p):
            b = bufs[p]
            d = dict(sbuf=b[0], dbuf=b[1], gidx=b[2], rowbuf=b[3],
                     widebuf=b[4])
            i = 5
            if not ssrc:
                d["av"] = b[i]; i += 1
            if not sdst:
                d["dcl"] = b[i]; d["bv"] = b[i + 1]; i += 2
            d["dscat"] = b[i]; i += 1
            d["sem_i"] = b[i]; d["sem_g"] = b[i + 1]; d["sem_s"] = b[i + 2]
            return d

        def issue_idx(p, chunk):
            b = parts(p)
            base = chunk * k_e
            pltpu.async_copy(src_hbm.at[pl.ds(base, k_e)], b["sbuf"],
                             b["sem_i"])
            pltpu.async_copy(dst_hbm.at[pl.ds(base, k_e)], b["dbuf"],
                             b["sem_i"])

        def wait_idx(p, chunk):
            b = parts(p)
            base = chunk * k_e
            pltpu.make_async_copy(src_hbm.at[pl.ds(base, k_e)], b["sbuf"],
                                  b["sem_i"]).wait()
            pltpu.make_async_copy(dst_hbm.at[pl.ds(base, k_e)], b["dbuf"],
                                  b["sem_i"]).wait()

        def prep(p):
            b = parts(p)

            @plsc.parallel_loop(0, k_e // 16, unroll=4)
            def pbody(j):
                sl = pl.ds(j * 16, 16)
                b["gidx"][sl] = b["sbuf"][sl] * 2 + c
                if not sdst:
                    b["dcl"][sl] = jnp.minimum(b["dbuf"][sl], n_dst - 1)

        def issue_gat(p):
            b = parts(p)
            pltpu.async_copy(h2_hbm.at[b["gidx"]], b["rowbuf"], b["sem_g"])
            if not ssrc:
                pltpu.async_copy(asrc_hbm.at[b["sbuf"]], b["av"], b["sem_g"])
            if not sdst:
                pltpu.async_copy(adst_hbm.at[b["dcl"]], b["bv"], b["sem_g"])

        def wait_gat(p):
            b = parts(p)
            pltpu.make_async_copy(h2_hbm.at[b["gidx"]], b["rowbuf"],
                                  b["sem_g"]).wait()
            if not ssrc:
                pltpu.make_async_copy(asrc_hbm.at[b["sbuf"]], b["av"],
                                      b["sem_g"]).wait()
            if not sdst:
                pltpu.make_async_copy(adst_hbm.at[b["dcl"]], b["bv"],
                                      b["sem_g"]).wait()

        def wait_scat(p):
            b = parts(p)
            pltpu.make_async_copy(b["widebuf"], num_sh.at[b["dscat"]],
                                  b["sem_s"]).wait()

        def scale_scatter(p):
            b = parts(p)
            wait_scat(p)

            @plsc.parallel_loop(0, k_e // 16, unroll=2)
            def sbody(j):
                sl = pl.ds(j * 16, 16)
                if ssrc:
                    aval = plsc.load_gather(asrc_v, [b["sbuf"][sl]])
                else:
                    aval = b["av"][sl]
                if sdst:
                    dv = jnp.minimum(b["dbuf"][sl], n_dst - 1)
                    bval = plsc.load_gather(adst_v, [dv])
                else:
                    bval = b["bv"][sl]
                al = aval + bval
                al = jnp.where(al >= 0.0, al, 0.2 * al)
                wbuf[sl] = jnp.exp(al)
                for l in range(16):
                    e = j * 16 + l
                    wb = plsc.load_gather(
                        wbuf, [jnp.full((16,), e, jnp.int32)])
                    for q in range(4):
                        ql = pl.ds(q * 16, 16)
                        b["widebuf"][e, ql] = b["rowbuf"][e, ql] * wb
                    b["widebuf"][e, pl.ds(64, 16)] = jnp.where(
                        lanes == 0, wb, 0.0)
                b["dscat"][sl] = b["dbuf"][sl]
            pltpu.async_copy(b["widebuf"], num_sh.at[b["dscat"]], b["sem_s"],
                             add=True)

        jrow = jnp.full((16,), n_dst, jnp.int32)

        def wzero(p):
            b = parts(p)

            def wzbody(r, carry):
                for q in range(5):
                    b["widebuf"][r, pl.ds(q * 16, 16)] = zero16
                return carry

            lax.fori_loop(0, k_e, wzbody, 0)

            def dzbody(j, carry):
                b["dscat"][pl.ds(j * 16, 16)] = jrow
                return carry

            lax.fori_loop(0, k_e // 16, dzbody, 0)

        wzero(0)
        wzero(1)
        zc = min(k_e, r16)
        while r16 % zc:
            zc //= 2
        wb0 = parts(0)["widebuf"]

        def zbody(i, carry):
            pltpu.sync_copy(wb0.at[pl.ds(0, zc)],
                            num_sh.at[pl.ds(s * r16 + i * zc, zc)])
            return carry

        lax.fori_loop(0, r16 // zc, zbody, 0)
        pltpu.async_copy(parts(0)["widebuf"], num_sh.at[parts(0)["dscat"]],
                         parts(0)["sem_s"], add=True)
        pltpu.async_copy(parts(1)["widebuf"], num_sh.at[parts(1)["dscat"]],
                         parts(1)["sem_s"], add=True)
        if ssrc:
            pltpu.sync_copy(asrc_hbm, asrc_v)
        if sdst:
            pltpu.sync_copy(adst_hbm, adst_v)
        plsc.subcore_barrier()

        c0 = s * cw
        issue_idx(0, c0)
        wait_idx(0, c0)
        prep(0)
        issue_gat(0)

        def pair_body(i2, carry):
            a = c0 + 2 * i2
            nxt = jnp.minimum(a + 2, c0 + cw - 1)
            issue_idx(1, a + 1)
            wait_idx(1, a + 1)
            prep(1)
            issue_gat(1)
            wait_gat(0)
            scale_scatter(0)
            issue_idx(0, nxt)
            wait_idx(0, nxt)
            prep(0)
            issue_gat(0)
            wait_gat(1)
            scale_scatter(1)
            return carry

        lax.fori_loop(0, cw // 2, pair_body, 0)
        wait_gat(0)
        wait_scat(0)
        wait_scat(1)
        plsc.subcore_barrier()

        def drain_body(i, carry):
            r0 = s * r16 + i * cr
            pltpu.sync_copy(num_sh.at[pl.ds(r0, cr)], drainbuf)
            for r in range(cr):
                den = plsc.load_gather(
                    drainbuf,
                    [jnp.full((16,), r, jnp.int32),
                     jnp.full((16,), 64, jnp.int32)],
                )
                m = den > 0.0
                for q in range(4):
                    ql = pl.ds(q * 16, 16)
                    obuf[r, ql] = jnp.where(m, drainbuf[r, ql] / den, 0.0)
            pltpu.sync_copy(obuf, out_hbm.at[pl.ds(r0, cr), pl.ds(c * 64, 64)])
            return carry

        lax.fori_loop(0, r16 // cr, drain_body, 0)

    return k(h2, a_src, a_dst, src_e, dst_e)


def _readout_sc(hd, hc, drug1, drug2, cell):
    b_n = drug1.shape[0]
    rb = b_n // 32
    mesh = plsc.VectorSubcoreMesh(core_axis_name="c", subcore_axis_name="s")
    out_t = jax.ShapeDtypeStruct((b_n, 128), jnp.float32)

    @functools.partial(
        pl.kernel,
        out_type=(out_t, out_t, out_t),
        mesh=mesh,
        compiler_params=pltpu.CompilerParams(use_tc_tiling_on_sc=False, needs_layout_passes=False),
        scratch_types=[
            pltpu.VMEM((rb,), jnp.int32),
            pltpu.VMEM((rb, 128), jnp.float32),
            pltpu.SemaphoreType.DMA,
        ],
    )
    def k(hd_hbm, hc_hbm, d1_hbm, d2_hbm, cl_hbm, o1, o2, o3, idx_v, buf, sem):
        wid = lax.axis_index("s") * 2 + lax.axis_index("c")
        base = wid * rb
        for idx_hbm, tab_hbm, out_hbm in (
            (d1_hbm, hd_hbm, o1),
            (d2_hbm, hd_hbm, o2),
            (cl_hbm, hc_hbm, o3),
        ):
            pltpu.sync_copy(idx_hbm.at[pl.ds(base, rb)], idx_v)
            pltpu.async_copy(tab_hbm.at[idx_v], buf, sem).wait()
            pltpu.sync_copy(buf, out_hbm.at[pl.ds(base, rb)])

    return k(hd, hc, drug1, drug2, cell)


def _pad_edges(src, dst, n_dst, k_e):
    e = src.shape[0]
    e_pad = _round_up(e, _NS * k_e * 2)
    pad = e_pad - e
    src = jnp.concatenate([src, jnp.zeros((pad,), jnp.int32)])
    dst = jnp.concatenate([dst, jnp.full((pad,), n_dst, jnp.int32)])
    return src, dst


def _score_cols(w_l, specs):
    cols = [w_l[r] @ v[r] for (r, v) in specs]
    g = jnp.stack(cols, axis=1)
    return jnp.pad(g, ((0, 0), (0, 128 - g.shape[1])))


def _tail(plan):
    k_e, ssrc, sdst, n_pad = plan
    return (n_pad, k_e, ssrc, sdst)


def kernel(x_drug, x_protein, x_cell, edge_index_dd, edge_index_dp,
           edge_index_rev_dp, edge_index_pp, edge_index_cp, edge_index_rev_cp,
           drug1, drug2, cell, drug_table, protein_table, cell_table,
           W0, as0, ad0, b0, W1, as1, ad1, b1, cW1, cb1, cW2, cb2, cW3, cb3):
    nd = drug_table.shape[0]
    np_ = protein_table.shape[0]
    nc = cell_table.shape[0]
    pl_dd = _plan(nd, nd)
    pl_dp = _plan(nd, np_)
    pl_rdp = _plan(np_, nd)
    pl_pp = _plan(np_, np_)
    pl_cp = _plan(nc, np_)
    pl_rcp = _plan(np_, nc)
    pad_d = pl_dd[3]
    pad_p = pl_dp[3]
    pad_c = pl_rcp[3]

    hd0 = jnp.take(drug_table, x_drug, axis=0)
    hp0 = jnp.take(protein_table, x_protein, axis=0)
    hc0 = jnp.take(cell_table, x_cell, axis=0)

    ar_d = jnp.arange(nd, dtype=jnp.int32)
    ar_p = jnp.arange(np_, dtype=jnp.int32)
    s_dd, d_dd = _pad_edges(
        jnp.concatenate([edge_index_dd[0], ar_d]),
        jnp.concatenate([edge_index_dd[1], ar_d]), nd, pl_dd[0])
    s_dp, d_dp = _pad_edges(edge_index_dp[0], edge_index_dp[1], np_, pl_dp[0])
    s_rdp, d_rdp = _pad_edges(edge_index_rev_dp[0], edge_index_rev_dp[1], nd, pl_rdp[0])
    s_pp, d_pp = _pad_edges(
        jnp.concatenate([edge_index_pp[0], ar_p]),
        jnp.concatenate([edge_index_pp[1], ar_p]), np_, pl_pp[0])
    s_cp, d_cp = _pad_edges(edge_index_cp[0], edge_index_cp[1], np_, pl_cp[0])
    s_rcp, d_rcp = _pad_edges(edge_index_rev_cp[0], edge_index_rev_cp[1], nc, pl_rcp[0])

    zbias = jnp.zeros((1, 128), jnp.float32)

    gd0 = _score_cols(W0, [(0, as0), (1, as0), (0, ad0), (2, ad0)])
    gp0 = _score_cols(
        W0, [(2, as0), (3, as0), (5, as0), (1, ad0), (3, ad0), (4, ad0)])
    gc0 = _score_cols(W0, [(4, as0), (5, ad0)])
    yd = _proj_tc([hd0], zbias, jnp.stack([W0[0], W0[1], gd0]), nd, False)
    yp = _proj_tc([hp0], zbias,
                  jnp.stack([W0[2], W0[3], W0[5], gp0]), np_, False)
    yc = _proj_tc([hc0], zbias, jnp.stack([W0[4], gc0]), nc, False)

    sd = yd[2]
    sp = yp[3]
    sc = yc[1]

    def h2(y):
        return y.reshape(2 * y.shape[0], 64)

    od_dd = _gat_edge_sc(h2(yd[0]), sd[:, 0], sd[:, 2], s_dd, d_dd, nd, *_tail(pl_dd))
    op_dp = _gat_edge_sc(h2(yd[1]), sd[:, 1], sp[:, 3], s_dp, d_dp, np_, *_tail(pl_dp))
    od_rdp = _gat_edge_sc(
        h2(yp[0]), sp[:, 0], sd[:, 3], s_rdp, d_rdp, nd, *_tail(pl_rdp))
    op_pp = _gat_edge_sc(h2(yp[1]), sp[:, 1], sp[:, 4], s_pp, d_pp, np_, *_tail(pl_pp))
    op_cp = _gat_edge_sc(h2(yc[0]), sc[:, 0], sp[:, 5], s_cp, d_cp, np_, *_tail(pl_cp))
    oc_rcp = _gat_edge_sc(
        h2(yp[2]), sp[:, 2], sc[:, 1], s_rcp, d_rcp, nc, *_tail(pl_rcp))

    gd1 = _score_cols(W1, [(0, as1), (0, ad1), (2, ad1)])
    gp1 = _score_cols(W1, [(2, as1), (5, as1)])
    gc1 = _score_cols(W1, [(5, ad1)])
    bias_d = (b0[0] + b0[2]).reshape(1, 128)
    bias_p = (b0[1] + b0[3] + b0[4]).reshape(1, 128)
    bias_c = b0[5].reshape(1, 128)
    yd1 = _proj_tc([od_dd, od_rdp], bias_d, jnp.stack([W1[0], gd1]), nd, True)
    yp1 = _proj_tc([op_dp, op_pp, op_cp], bias_p,
                   jnp.stack([W1[2], W1[5], gp1]), np_, True)
    yc1 = _proj_tc([oc_rcp], bias_c, jnp.stack([gc1]), nc, True)

    sd1 = yd1[1]
    sp1 = yp1[2]
    sc1 = yc1[0]
    od_dd1 = _gat_edge_sc(
        h2(yd1[0]), sd1[:, 0], sd1[:, 1], s_dd, d_dd, nd, *_tail(pl_dd))
    od_rdp1 = _gat_edge_sc(
        h2(yp1[0]), sp1[:, 0], sd1[:, 2], s_rdp, d_rdp, nd, *_tail(pl_rdp))
    oc_rcp1 = _gat_edge_sc(
        h2(yp1[1]), sp1[:, 1], sc1[:, 0], s_rcp, d_rcp, nc, *_tail(pl_rcp))

    hd_fin = _finalize_tc(
        [od_dd1, od_rdp1], (b1[0] + b1[2]).reshape(1, 128), pad_d)
    hc_fin = _finalize_tc([oc_rcp1], b1[5].reshape(1, 128), pad_c)
    g1, g2, g3 = _readout_sc(hd_fin, hc_fin, drug1, drug2, cell)

    w3p = jnp.pad(cW3, ((0, 0), (0, 126)))
    b3p = jnp.pad(cb3, (0, 126)).reshape(1, 128)
    out = _mlp_tc(g1, g2, g3, cW1[:128], cW1[128:256], cW1[256:384],
                  cb1.reshape(1, 768), cW2, cb2.reshape(1, 256), w3p, b3p)
    return out[:, :2]


# speedup vs baseline: 15.7487x; 1.0059x over previous
import functools

import jax
import jax.numpy as jnp
from jax import lax
from jax.experimental import pallas as pl
from jax.experimental.pallas import tpu as pltpu
from jax.experimental.pallas import tpu_sc as plsc

_NS = 16
_K = 128


def _round_up(x, m):
    return ((x + m - 1) // m) * m


def _pick_bn(n, cap=2048):
    bn = 8
    for d in range(8, min(n, cap) + 1, 8):
        if n % d == 0:
            bn = d
    return bn


def _proj_tc(parts, bias_row, wst, n, relu):
    j_n = wst.shape[0]
    bn = _pick_bn(n)
    nb = n // bn
    npart = len(parts)

    def body(*refs):
        part_refs = refs[:npart]
        bias_ref = refs[npart]
        w_ref = refs[npart + 1]
        out_ref = refs[npart + 2]
        x = part_refs[0][...]
        for p in part_refs[1:]:
            x = x + p[...]
        x = x + bias_ref[...]
        if relu:
            x = jnp.maximum(x, 0.0)
        out_ref[0] = jnp.dot(x, w_ref[0], preferred_element_type=jnp.float32)

    in_specs = (
        [pl.BlockSpec((bn, 128), lambda i, j: (i, 0)) for _ in parts]
        + [pl.BlockSpec((1, 128), lambda i, j: (0, 0))]
        + [pl.BlockSpec((1, 128, 128), lambda i, j: (j, 0, 0))]
    )
    return pl.pallas_call(
        body,
        grid=(nb, j_n),
        in_specs=in_specs,
        out_specs=pl.BlockSpec((1, bn, 128), lambda i, j: (j, i, 0)),
        out_shape=jax.ShapeDtypeStruct((j_n, n, 128), jnp.float32),
    )(*parts, bias_row, wst)


def _finalize_tc(parts, bias_row, n_pad):
    bn = 1024 if n_pad % 1024 == 0 else 256
    npart = len(parts)

    def body(*refs):
        x = refs[0][...]
        for p in refs[1:npart]:
            x = x + p[...]
        x = x + refs[npart][...]
        refs[npart + 1][...] = jnp.maximum(x, 0.0)

    in_specs = [pl.BlockSpec((bn, 128), lambda i: (i, 0)) for _ in parts] + [
        pl.BlockSpec((1, 128), lambda i: (0, 0))
    ]
    return pl.pallas_call(
        body,
        grid=(n_pad // bn,),
        in_specs=in_specs,
        out_specs=pl.BlockSpec((bn, 128), lambda i: (i, 0)),
        out_shape=jax.ShapeDtypeStruct((n_pad, 128), jnp.float32),
    )(*parts, bias_row)


def _mlp_tc(g1, g2, g3, w1a, w1b, w1c, b1, w2, b2, w3, b3):
    b_n = g1.shape[0]
    bm = 512

    def body(g1r, g2r, g3r, w1ar, w1br, w1cr, b1r, w2r, b2r, w3r, b3r, outr):
        h = jnp.dot(g1r[...], w1ar[...], preferred_element_type=jnp.float32)
        h = h + jnp.dot(g2r[...], w1br[...], preferred_element_type=jnp.float32)
        h = h + jnp.dot(g3r[...], w1cr[...], preferred_element_type=jnp.float32)
        h = jnp.maximum(h + b1r[...], 0.0)
        h = jnp.maximum(
            jnp.dot(h, w2r[...], preferred_element_type=jnp.float32) + b2r[...],
            0.0,
        )
        outr[...] = (
            jnp.dot(h, w3r[...], preferred_element_type=jnp.float32) + b3r[...]
        )

    def full(shape):
        return pl.BlockSpec(shape, lambda i: tuple(0 for _ in shape))

    return pl.pallas_call(
        body,
        grid=(b_n // bm,),
        in_specs=[
            pl.BlockSpec((bm, 128), lambda i: (i, 0)),
            pl.BlockSpec((bm, 128), lambda i: (i, 0)),
            pl.BlockSpec((bm, 128), lambda i: (i, 0)),
            full((128, 768)),
            full((128, 768)),
            full((128, 768)),
            full((1, 768)),
            full((768, 256)),
            full((1, 256)),
            full((256, 128)),
            full((1, 128)),
        ],
        out_specs=pl.BlockSpec((bm, 128), lambda i: (i, 0)),
        out_shape=jax.ShapeDtypeStruct((b_n, 128), jnp.float32),
    )(g1, g2, g3, w1a, w1b, w1c, b1, w2, b2, w3, b3)


_POOL = 2097151


def _plan(n_src, n_dst):
    n_pad = _round_up(n_dst + 1, 512)
    free_tile = (_POOL - n_pad * 80) // _NS - 4096
    for k in (256, 128, 64):
        for ssrc in (True, False):
            for sdst in (True, False):
                use = k * (6 + 2 + 128 + 160) + 640 + 1280 + 1024
                use += (n_src if ssrc else 2 * k)
                use += (n_dst if sdst else 4 * k)
                if use <= free_tile:
                    return k, ssrc, sdst, n_pad
    return 64, False, False, n_pad


def _gat_edge_sc(h2, a_src, a_dst, src_e, dst_e, n_dst, n_pad, k_e, ssrc, sdst):
    n_src = a_src.shape[0]
    e_pad = src_e.shape[0]
    cw = e_pad // (_NS * k_e)
    r16 = n_pad // _NS
    cr = 32
    mesh = plsc.VectorSubcoreMesh(core_axis_name="c", subcore_axis_name="s")

    scratch = []
    for _ in range(2):
        scratch += [pltpu.VMEM((k_e,), jnp.int32)] * 3
        scratch += [pltpu.VMEM((k_e, 64), jnp.float32)]
        scratch += [pltpu.VMEM((k_e, 80), jnp.float32)]
        if not ssrc:
            scratch += [pltpu.VMEM((k_e,), jnp.float32)]
        if not sdst:
            scratch += [pltpu.VMEM((k_e,), jnp.int32)]
            scratch += [pltpu.VMEM((k_e,), jnp.float32)]
        scratch += [pltpu.VMEM((k_e,), jnp.int32)]
        scratch += [pltpu.SemaphoreType.DMA] * 3
    nper = 5 + (0 if ssrc else 1) + (0 if sdst else 2) + 4
    if ssrc:
        scratch += [pltpu.VMEM((n_src,), jnp.float32)]
    if sdst:
        scratch += [pltpu.VMEM((n_dst,), jnp.float32)]
    scratch += [
        pltpu.VMEM((k_e,), jnp.float32),
        pltpu.VMEM((cr, 80), jnp.float32),
        pltpu.VMEM((cr, 64), jnp.float32),
        pltpu.VMEM_SHARED((n_pad, 80), jnp.float32),
    ]

    @functools.partial(
        pl.kernel,
        out_type=jax.ShapeDtypeStruct((n_pad, 128), jnp.float32),
        mesh=mesh,
        compiler_params=pltpu.CompilerParams(
            use_tc_tiling_on_sc=False, needs_layout_passes=False),
        scratch_types=scratch,
    )
    def k(h2_hbm, asrc_hbm, adst_hbm, src_hbm, dst_hbm, out_hbm, *sc):
        bufs = {0: sc[:nper], 1: sc[nper:2 * nper]}
        rest = list(sc[2 * nper:])
        asrc_v = rest.pop(0) if ssrc else None
        adst_v = rest.pop(0) if sdst else None
        wbuf, drainbuf, obuf, num_sh = rest
        c = lax.axis_index("c")
        s = lax.axis_index("s")
        zero16 = jnp.zeros((16,), jnp.float32)
        lanes = jnp.arange(16, dtype=jnp.int32)

        def parts(p):
            b = bufs[p]
            d = dict(sbuf=b[0], dbuf=b[1], gidx=b[2], rowbuf=b[3],
                     widebuf=b[4])
            i = 5
            if not ssrc:
                d["av"] = b[i]; i += 1
            if not sdst:
                d["dcl"] = b[i]; d["bv"] = b[i + 1]; i += 2
            d["dscat"] = b[i]; i += 1
            d["sem_i"] = b[i]; d["sem_g"] = b[i + 1]; d["sem_s"] = b[i + 2]
            return d

        def issue_idx(p, chunk):
            b = parts(p)
            base = chunk * k_e
            pltpu.async_copy(src_hbm.at[pl.ds(base, k_e)], b["sbuf"],
                             b["sem_i"])
            pltpu.async_copy(dst_hbm.at[pl.ds(base, k_e)], b["dbuf"],
                             b["sem_i"])

        def wait_idx(p, chunk):
            b = parts(p)
            base = chunk * k_e
            pltpu.make_async_copy(src_hbm.at[pl.ds(base, k_e)], b["sbuf"],
                                  b["sem_i"]).wait()
            pltpu.make_async_copy(dst_hbm.at[pl.ds(base, k_e)], b["dbuf"],
                                  b["sem_i"]).wait()

        def prep(p):
            b = parts(p)

            @plsc.parallel_loop(0, k_e // 16, unroll=4)
            def pbody(j):
                sl = pl.ds(j * 16, 16)
                b["gidx"][sl] = b["sbuf"][sl] * 2 + c
                if not sdst:
                    b["dcl"][sl] = jnp.minimum(b["dbuf"][sl], n_dst - 1)

        def issue_gat(p):
            b = parts(p)
            pltpu.async_copy(h2_hbm.at[b["gidx"]], b["rowbuf"], b["sem_g"])
            if not ssrc:
                pltpu.async_copy(asrc_hbm.at[b["sbuf"]], b["av"], b["sem_g"])
            if not sdst:
                pltpu.async_copy(adst_hbm.at[b["dcl"]], b["bv"], b["sem_g"])

        def wait_gat(p):
            b = parts(p)
            pltpu.make_async_copy(h2_hbm.at[b["gidx"]], b["rowbuf"],
                                  b["sem_g"]).wait()
            if not ssrc:
                pltpu.make_async_copy(asrc_hbm.at[b["sbuf"]], b["av"],
                                      b["sem_g"]).wait()
            if not sdst:
                pltpu.make_async_copy(adst_hbm.at[b["dcl"]], b["bv"],
                                      b["sem_g"]).wait()

        def wait_scat(p):
            b = parts(p)
            pltpu.make_async_copy(b["widebuf"], num_sh.at[b["dscat"]],
                                  b["sem_s"]).wait()

        def scale_scatter(p):
            b = parts(p)
            wait_scat(p)

            @plsc.parallel_loop(0, k_e // 16, unroll=4)
            def sbody(j):
                sl = pl.ds(j * 16, 16)
                if ssrc:
                    aval = plsc.load_gather(asrc_v, [b["sbuf"][sl]])
                else:
                    aval = b["av"][sl]
                if sdst:
                    dv = jnp.minimum(b["dbuf"][sl], n_dst - 1)
                    bval = plsc.load_gather(adst_v, [dv])
                else:
                    bval = b["bv"][sl]
                al = aval + bval
                al = jnp.where(al >= 0.0, al, 0.2 * al)
                wbuf[sl] = jnp.exp(al)
                for l in range(16):
                    e = j * 16 + l
                    wb = plsc.load_gather(
                        wbuf, [jnp.full((16,), e, jnp.int32)])
                    for q in range(4):
                        ql = pl.ds(q * 16, 16)
                        b["widebuf"][e, ql] = b["rowbuf"][e, ql] * wb
                    b["widebuf"][e, pl.ds(64, 16)] = jnp.where(
                        lanes == 0, wb, 0.0)
                b["dscat"][sl] = b["dbuf"][sl]
            pltpu.async_copy(b["widebuf"], num_sh.at[b["dscat"]], b["sem_s"],
                             add=True)

        jrow = jnp.full((16,), n_dst, jnp.int32)

        def wzero(p):
            b = parts(p)

            def wzbody(r, carry):
                for q in range(5):
                    b["widebuf"][r, pl.ds(q * 16, 16)] = zero16
                return carry

            lax.fori_loop(0, k_e, wzbody, 0)

            def dzbody(j, carry):
                b["dscat"][pl.ds(j * 16, 16)] = jrow
                return carry

            lax.fori_loop(0, k_e // 16, dzbody, 0)

        wzero(0)
        wzero(1)
        zc = min(k_e, r16)
        while r16 % zc:
            zc //= 2
        wb0 = parts(0)["widebuf"]

        def zbody(i, carry):
            pltpu.sync_copy(wb0.at[pl.ds(0, zc)],
                            num_sh.at[pl.ds(s * r16 + i * zc, zc)])
            return carry

        lax.fori_loop(0, r16 // zc, zbody, 0)
        pltpu.async_copy(parts(0)["widebuf"], num_sh.at[parts(0)["dscat"]],
                         parts(0)["sem_s"], add=True)
        pltpu.async_copy(parts(1)["widebuf"], num_sh.at[parts(1)["dscat"]],
                         parts(1)["sem_s"], add=True)
        if ssrc:
            pltpu.sync_copy(asrc_hbm, asrc_v)
        if sdst:
            pltpu.sync_copy(adst_hbm, adst_v)
        plsc.subcore_barrier()

        c0 = s * cw
        issue_idx(0, c0)
        wait_idx(0, c0)
        prep(0)
        issue_gat(0)

        def pair_body(i2, carry):
            a = c0 + 2 * i2
            nxt = jnp.minimum(a + 2, c0 + cw - 1)
            issue_idx(1, a + 1)
            wait_idx(1, a + 1)
            prep(1)
            issue_gat(1)
            wait_gat(0)
            scale_scatter(0)
            issue_idx(0, nxt)
            wait_idx(0, nxt)
            prep(0)
            issue_gat(0)
            wait_gat(1)
            scale_scatter(1)
            return carry

        lax.fori_loop(0, cw // 2, pair_body, 0)
        wait_gat(0)
        wait_scat(0)
        wait_scat(1)
        plsc.subcore_barrier()

        def drain_body(i, carry):
            r0 = s * r16 + i * cr
            pltpu.sync_copy(num_sh.at[pl.ds(r0, cr)], drainbuf)
            for r in range(cr):
                den = plsc.load_gather(
                    drainbuf,
                    [jnp.full((16,), r, jnp.int32),
                     jnp.full((16,), 64, jnp.int32)],
                )
                m = den > 0.0
                for q in range(4):
                    ql = pl.ds(q * 16, 16)
                    obuf[r, ql] = jnp.where(m, drainbuf[r, ql] / den, 0.0)
            pltpu.sync_copy(obuf, out_hbm.at[pl.ds(r0, cr), pl.ds(c * 64, 64)])
            return carry

        lax.fori_loop(0, r16 // cr, drain_body, 0)

    return k(h2, a_src, a_dst, src_e, dst_e)


def _readout_sc(hd, hc, drug1, drug2, cell):
    b_n = drug1.shape[0]
    rb = b_n // 32
    mesh = plsc.VectorSubcoreMesh(core_axis_name="c", subcore_axis_name="s")
    out_t = jax.ShapeDtypeStruct((b_n, 128), jnp.float32)

    @functools.partial(
        pl.kernel,
        out_type=(out_t, out_t, out_t),
        mesh=mesh,
        compiler_params=pltpu.CompilerParams(use_tc_tiling_on_sc=False, needs_layout_passes=False),
        scratch_types=[
            pltpu.VMEM((rb,), jnp.int32),
            pltpu.VMEM((rb, 128), jnp.float32),
            pltpu.SemaphoreType.DMA,
        ],
    )
    def k(hd_hbm, hc_hbm, d1_hbm, d2_hbm, cl_hbm, o1, o2, o3, idx_v, buf, sem):
        wid = lax.axis_index("s") * 2 + lax.axis_index("c")
        base = wid * rb
        for idx_hbm, tab_hbm, out_hbm in (
            (d1_hbm, hd_hbm, o1),
            (d2_hbm, hd_hbm, o2),
            (cl_hbm, hc_hbm, o3),
        ):
            pltpu.sync_copy(idx_hbm.at[pl.ds(base, rb)], idx_v)
            pltpu.async_copy(tab_hbm.at[idx_v], buf, sem).wait()
            pltpu.sync_copy(buf, out_hbm.at[pl.ds(base, rb)])

    return k(hd, hc, drug1, drug2, cell)


def _pad_edges(src, dst, n_dst, k_e):
    e = src.shape[0]
    e_pad = _round_up(e, _NS * k_e * 2)
    pad = e_pad - e
    src = jnp.concatenate([src, jnp.zeros((pad,), jnp.int32)])
    dst = jnp.concatenate([dst, jnp.full((pad,), n_dst, jnp.int32)])
    return src, dst


def _score_cols(w_l, specs):
    cols = [w_l[r] @ v[r] for (r, v) in specs]
    g = jnp.stack(cols, axis=1)
    return jnp.pad(g, ((0, 0), (0, 128 - g.shape[1])))


def _tail(plan):
    k_e, ssrc, sdst, n_pad = plan
    return (n_pad, k_e, ssrc, sdst)


def kernel(x_drug, x_protein, x_cell, edge_index_dd, edge_index_dp,
           edge_index_rev_dp, edge_index_pp, edge_index_cp, edge_index_rev_cp,
           drug1, drug2, cell, drug_table, protein_table, cell_table,
           W0, as0, ad0, b0, W1, as1, ad1, b1, cW1, cb1, cW2, cb2, cW3, cb3):
    nd = drug_table.shape[0]
    np_ = protein_table.shape[0]
    nc = cell_table.shape[0]
    pl_dd = _plan(nd, nd)
    pl_dp = _plan(nd, np_)
    pl_rdp = _plan(np_, nd)
    pl_pp = _plan(np_, np_)
    pl_cp = _plan(nc, np_)
    pl_rcp = _plan(np_, nc)
    pad_d = pl_dd[3]
    pad_p = pl_dp[3]
    pad_c = pl_rcp[3]

    hd0 = jnp.take(drug_table, x_drug, axis=0)
    hp0 = jnp.take(protein_table, x_protein, axis=0)
    hc0 = jnp.take(cell_table, x_cell, axis=0)

    ar_d = jnp.arange(nd, dtype=jnp.int32)
    ar_p = jnp.arange(np_, dtype=jnp.int32)
    s_dd, d_dd = _pad_edges(
        jnp.concatenate([edge_index_dd[0], ar_d]),
        jnp.concatenate([edge_index_dd[1], ar_d]), nd, pl_dd[0])
    s_dp, d_dp = _pad_edges(edge_index_dp[0], edge_index_dp[1], np_, pl_dp[0])
    s_rdp, d_rdp = _pad_edges(edge_index_rev_dp[0], edge_index_rev_dp[1], nd, pl_rdp[0])
    s_pp, d_pp = _pad_edges(
        jnp.concatenate([edge_index_pp[0], ar_p]),
        jnp.concatenate([edge_index_pp[1], ar_p]), np_, pl_pp[0])
    s_cp, d_cp = _pad_edges(edge_index_cp[0], edge_index_cp[1], np_, pl_cp[0])
    s_rcp, d_rcp = _pad_edges(edge_index_rev_cp[0], edge_index_rev_cp[1], nc, pl_rcp[0])

    zbias = jnp.zeros((1, 128), jnp.float32)

    gd0 = _score_cols(W0, [(0, as0), (1, as0), (0, ad0), (2, ad0)])
    gp0 = _score_cols(
        W0, [(2, as0), (3, as0), (5, as0), (1, ad0), (3, ad0), (4, ad0)])
    gc0 = _score_cols(W0, [(4, as0), (5, ad0)])
    yd = _proj_tc([hd0], zbias, jnp.stack([W0[0], W0[1], gd0]), nd, False)
    yp = _proj_tc([hp0], zbias,
                  jnp.stack([W0[2], W0[3], W0[5], gp0]), np_, False)
    yc = _proj_tc([hc0], zbias, jnp.stack([W0[4], gc0]), nc, False)

    sd = yd[2]
    sp = yp[3]
    sc = yc[1]

    def h2(y):
        return y.reshape(2 * y.shape[0], 64)

    od_dd = _gat_edge_sc(h2(yd[0]), sd[:, 0], sd[:, 2], s_dd, d_dd, nd, *_tail(pl_dd))
    op_dp = _gat_edge_sc(h2(yd[1]), sd[:, 1], sp[:, 3], s_dp, d_dp, np_, *_tail(pl_dp))
    od_rdp = _gat_edge_sc(
        h2(yp[0]), sp[:, 0], sd[:, 3], s_rdp, d_rdp, nd, *_tail(pl_rdp))
    op_pp = _gat_edge_sc(h2(yp[1]), sp[:, 1], sp[:, 4], s_pp, d_pp, np_, *_tail(pl_pp))
    op_cp = _gat_edge_sc(h2(yc[0]), sc[:, 0], sp[:, 5], s_cp, d_cp, np_, *_tail(pl_cp))
    oc_rcp = _gat_edge_sc(
        h2(yp[2]), sp[:, 2], sc[:, 1], s_rcp, d_rcp, nc, *_tail(pl_rcp))

    gd1 = _score_cols(W1, [(0, as1), (0, ad1), (2, ad1)])
    gp1 = _score_cols(W1, [(2, as1), (5, as1)])
    gc1 = _score_cols(W1, [(5, ad1)])
    bias_d = (b0[0] + b0[2]).reshape(1, 128)
    bias_p = (b0[1] + b0[3] + b0[4]).reshape(1, 128)
    bias_c = b0[5].reshape(1, 128)
    yd1 = _proj_tc([od_dd, od_rdp], bias_d, jnp.stack([W1[0], gd1]), nd, True)
    yp1 = _proj_tc([op_dp, op_pp, op_cp], bias_p,
                   jnp.stack([W1[2], W1[5], gp1]), np_, True)
    yc1 = _proj_tc([oc_rcp], bias_c, jnp.stack([gc1]), nc, True)

    sd1 = yd1[1]
    sp1 = yp1[2]
    sc1 = yc1[0]
    od_dd1 = _gat_edge_sc(
        h2(yd1[0]), sd1[:, 0], sd1[:, 1], s_dd, d_dd, nd, *_tail(pl_dd))
    od_rdp1 = _gat_edge_sc(
        h2(yp1[0]), sp1[:, 0], sd1[:, 2], s_rdp, d_rdp, nd, *_tail(pl_rdp))
    oc_rcp1 = _gat_edge_sc(
        h2(yp1[1]), sp1[:, 1], sc1[:, 0], s_rcp, d_rcp, nc, *_tail(pl_rcp))

    hd_fin = _finalize_tc(
        [od_dd1, od_rdp1], (b1[0] + b1[2]).reshape(1, 128), pad_d)
    hc_fin = _finalize_tc([oc_rcp1], b1[5].reshape(1, 128), pad_c)
    g1, g2, g3 = _readout_sc(hd_fin, hc_fin, drug1, drug2, cell)

    w3p = jnp.pad(cW3, ((0, 0), (0, 126)))
    b3p = jnp.pad(cb3, (0, 126)).reshape(1, 128)
    out = _mlp_tc(g1, g2, g3, cW1[:128], cW1[128:256], cW1[256:384],
                  cb1.reshape(1, 768), cW2, cb2.reshape(1, 256), w3p, b3p)
    return out[:, :2]


# speedup vs baseline: 16.4576x; 1.0450x over previous
import functools

import jax
import jax.numpy as jnp
from jax import lax
from jax.experimental import pallas as pl
from jax.experimental.pallas import tpu as pltpu
from jax.experimental.pallas import tpu_sc as plsc

_NS = 16
_K = 128


def _round_up(x, m):
    return ((x + m - 1) // m) * m


def _pick_bn(n, cap=2048):
    bn = 8
    for d in range(8, min(n, cap) + 1, 8):
        if n % d == 0:
            bn = d
    return bn


def _proj_tc(parts, bias_row, wst, n, relu):
    j_n = wst.shape[0]
    bn = _pick_bn(n)
    nb = n // bn
    npart = len(parts)

    def body(*refs):
        part_refs = refs[:npart]
        bias_ref = refs[npart]
        w_ref = refs[npart + 1]
        out_ref = refs[npart + 2]
        x = part_refs[0][...]
        for p in part_refs[1:]:
            x = x + p[...]
        x = x + bias_ref[...]
        if relu:
            x = jnp.maximum(x, 0.0)
        out_ref[0] = jnp.dot(x, w_ref[0], preferred_element_type=jnp.float32)

    in_specs = (
        [pl.BlockSpec((bn, 128), lambda i, j: (i, 0)) for _ in parts]
        + [pl.BlockSpec((1, 128), lambda i, j: (0, 0))]
        + [pl.BlockSpec((1, 128, 128), lambda i, j: (j, 0, 0))]
    )
    return pl.pallas_call(
        body,
        grid=(nb, j_n),
        in_specs=in_specs,
        out_specs=pl.BlockSpec((1, bn, 128), lambda i, j: (j, i, 0)),
        out_shape=jax.ShapeDtypeStruct((j_n, n, 128), jnp.float32),
    )(*parts, bias_row, wst)


def _finalize_tc(parts, bias_row, n_pad):
    bn = 1024 if n_pad % 1024 == 0 else 256
    npart = len(parts)

    def body(*refs):
        x = refs[0][...]
        for p in refs[1:npart]:
            x = x + p[...]
        x = x + refs[npart][...]
        refs[npart + 1][...] = jnp.maximum(x, 0.0)

    in_specs = [pl.BlockSpec((bn, 128), lambda i: (i, 0)) for _ in parts] + [
        pl.BlockSpec((1, 128), lambda i: (0, 0))
    ]
    return pl.pallas_call(
        body,
        grid=(n_pad // bn,),
        in_specs=in_specs,
        out_specs=pl.BlockSpec((bn, 128), lambda i: (i, 0)),
        out_shape=jax.ShapeDtypeStruct((n_pad, 128), jnp.float32),
    )(*parts, bias_row)


def _mlp_tc(g1, g2, g3, w1a, w1b, w1c, b1, w2, b2, w3, b3):
    b_n = g1.shape[0]
    bm = 512

    def body(g1r, g2r, g3r, w1ar, w1br, w1cr, b1r, w2r, b2r, w3r, b3r, outr):
        h = jnp.dot(g1r[...], w1ar[...], preferred_element_type=jnp.float32)
        h = h + jnp.dot(g2r[...], w1br[...], preferred_element_type=jnp.float32)
        h = h + jnp.dot(g3r[...], w1cr[...], preferred_element_type=jnp.float32)
        h = jnp.maximum(h + b1r[...], 0.0)
        h = jnp.maximum(
            jnp.dot(h, w2r[...], preferred_element_type=jnp.float32) + b2r[...],
            0.0,
        )
        outr[...] = (
            jnp.dot(h, w3r[...], preferred_element_type=jnp.float32) + b3r[...]
        )

    def full(shape):
        return pl.BlockSpec(shape, lambda i: tuple(0 for _ in shape))

    return pl.pallas_call(
        body,
        grid=(b_n // bm,),
        in_specs=[
            pl.BlockSpec((bm, 128), lambda i: (i, 0)),
            pl.BlockSpec((bm, 128), lambda i: (i, 0)),
            pl.BlockSpec((bm, 128), lambda i: (i, 0)),
            full((128, 768)),
            full((128, 768)),
            full((128, 768)),
            full((1, 768)),
            full((768, 256)),
            full((1, 256)),
            full((256, 128)),
            full((1, 128)),
        ],
        out_specs=pl.BlockSpec((bm, 128), lambda i: (i, 0)),
        out_shape=jax.ShapeDtypeStruct((b_n, 128), jnp.float32),
    )(g1, g2, g3, w1a, w1b, w1c, b1, w2, b2, w3, b3)


_POOL = 2097151


def _plan(n_src, n_dst):
    n_pad = _round_up(n_dst + 1, 512)
    free_tile = (_POOL - n_pad * 80) // _NS - 4096
    for k in (256, 128, 64):
        for ssrc in (True, False):
            for sdst in (True, False):
                use = k * (6 + 2 + 128 + 160) + 640 + 1280 + 1024
                use += (n_src if ssrc else 2 * k)
                use += (n_dst if sdst else 4 * k)
                if use <= free_tile:
                    return k, ssrc, sdst, n_pad
    return 64, False, False, n_pad


def _gat_edge_sc(h2, a_src, a_dst, src_e, dst_e, n_dst, n_pad, k_e, ssrc, sdst):
    n_src = a_src.shape[0]
    e_pad = src_e.shape[0]
    cw = e_pad // (_NS * k_e)
    r16 = n_pad // _NS
    cr = 32
    mesh = plsc.VectorSubcoreMesh(core_axis_name="c", subcore_axis_name="s")

    scratch = []
    for _ in range(2):
        scratch += [pltpu.VMEM((k_e,), jnp.int32)] * 3
        scratch += [pltpu.VMEM((k_e, 64), jnp.float32)]
        scratch += [pltpu.VMEM((k_e, 80), jnp.float32)]
        if not ssrc:
            scratch += [pltpu.VMEM((k_e,), jnp.float32)]
        if not sdst:
            scratch += [pltpu.VMEM((k_e,), jnp.int32)]
            scratch += [pltpu.VMEM((k_e,), jnp.float32)]
        scratch += [pltpu.VMEM((k_e,), jnp.int32)]
        scratch += [pltpu.SemaphoreType.DMA] * 3
    nper = 5 + (0 if ssrc else 1) + (0 if sdst else 2) + 4
    if ssrc:
        scratch += [pltpu.VMEM((n_src,), jnp.float32)]
    if sdst:
        scratch += [pltpu.VMEM((n_dst,), jnp.float32)]
    scratch += [
        pltpu.VMEM((k_e,), jnp.float32),
        pltpu.VMEM((cr, 80), jnp.float32),
        pltpu.VMEM((cr, 64), jnp.float32),
        pltpu.VMEM_SHARED((n_pad, 80), jnp.float32),
    ]

    @functools.partial(
        pl.kernel,
        out_type=jax.ShapeDtypeStruct((n_pad, 128), jnp.float32),
        mesh=mesh,
        compiler_params=pltpu.CompilerParams(
            use_tc_tiling_on_sc=False, needs_layout_passes=False),
        scratch_types=scratch,
    )
    def k(h2_hbm, asrc_hbm, adst_hbm, src_hbm, dst_hbm, out_hbm, *sc):
        bufs = {0: sc[:nper], 1: sc[nper:2 * nper]}
        rest = list(sc[2 * nper:])
        asrc_v = rest.pop(0) if ssrc else None
        adst_v = rest.pop(0) if sdst else None
        wbuf, drainbuf, obuf, num_sh = rest
        c = lax.axis_index("c")
        s = lax.axis_index("s")
        zero16 = jnp.zeros((16,), jnp.float32)
        lanes = jnp.arange(16, dtype=jnp.int32)

        def parts(p):
            b = bufs[p]
            d = dict(sbuf=b[0], dbuf=b[1], gidx=b[2], rowbuf=b[3],
                     widebuf=b[4])
            i = 5
            if not ssrc:
                d["av"] = b[i]; i += 1
            if not sdst:
                d["dcl"] = b[i]; d["bv"] = b[i + 1]; i += 2
            d["dscat"] = b[i]; i += 1
            d["sem_i"] = b[i]; d["sem_g"] = b[i + 1]; d["sem_s"] = b[i + 2]
            return d

        def issue_idx(p, chunk):
            b = parts(p)
            base = chunk * k_e
            pltpu.async_copy(src_hbm.at[pl.ds(base, k_e)], b["sbuf"],
                             b["sem_i"])
            pltpu.async_copy(dst_hbm.at[pl.ds(base, k_e)], b["dbuf"],
                             b["sem_i"])

        def wait_idx(p, chunk):
            b = parts(p)
            base = chunk * k_e
            pltpu.make_async_copy(src_hbm.at[pl.ds(base, k_e)], b["sbuf"],
                                  b["sem_i"]).wait()
            pltpu.make_async_copy(dst_hbm.at[pl.ds(base, k_e)], b["dbuf"],
                                  b["sem_i"]).wait()

        def prep(p):
            b = parts(p)

            @plsc.parallel_loop(0, k_e // 16, unroll=4)
            def pbody(j):
                sl = pl.ds(j * 16, 16)
                b["gidx"][sl] = b["sbuf"][sl] * 2 + c
                if not sdst:
                    b["dcl"][sl] = jnp.minimum(b["dbuf"][sl], n_dst - 1)

        def issue_gat(p):
            b = parts(p)
            pltpu.async_copy(h2_hbm.at[b["gidx"]], b["rowbuf"], b["sem_g"])
            if not ssrc:
                pltpu.async_copy(asrc_hbm.at[b["sbuf"]], b["av"], b["sem_g"])
            if not sdst:
                pltpu.async_copy(adst_hbm.at[b["dcl"]], b["bv"], b["sem_g"])

        def wait_gat(p):
            b = parts(p)
            pltpu.make_async_copy(h2_hbm.at[b["gidx"]], b["rowbuf"],
                                  b["sem_g"]).wait()
            if not ssrc:
                pltpu.make_async_copy(asrc_hbm.at[b["sbuf"]], b["av"],
                                      b["sem_g"]).wait()
            if not sdst:
                pltpu.make_async_copy(adst_hbm.at[b["dcl"]], b["bv"],
                                      b["sem_g"]).wait()

        def wait_scat(p):
            b = parts(p)
            pltpu.make_async_copy(b["widebuf"], num_sh.at[b["dscat"]],
                                  b["sem_s"]).wait()

        def scale_scatter(p):
            b = parts(p)
            wait_scat(p)

            @plsc.parallel_loop(0, k_e // 16, unroll=4)
            def sbody(j):
                sl = pl.ds(j * 16, 16)
                if ssrc:
                    aval = plsc.load_gather(asrc_v, [b["sbuf"][sl]])
                else:
                    aval = b["av"][sl]
                if sdst:
                    dv = jnp.minimum(b["dbuf"][sl], n_dst - 1)
                    bval = plsc.load_gather(adst_v, [dv])
                else:
                    bval = b["bv"][sl]
                al = aval + bval
                al = jnp.where(al >= 0.0, al, 0.2 * al)
                wv = jnp.exp(al)
                for l in range(16):
                    e = j * 16 + l
                    wb = lax.gather(
                        wv, jnp.full((16, 1), l, jnp.int32),
                        lax.GatherDimensionNumbers(
                            offset_dims=(), collapsed_slice_dims=(0,),
                            start_index_map=(0,)),
                        slice_sizes=(1,),
                        mode=lax.GatherScatterMode.PROMISE_IN_BOUNDS)
                    for q in range(4):
                        ql = pl.ds(q * 16, 16)
                        b["widebuf"][e, ql] = b["rowbuf"][e, ql] * wb
                    b["widebuf"][e, pl.ds(64, 16)] = jnp.where(
                        lanes == 0, wb, 0.0)
                b["dscat"][sl] = b["dbuf"][sl]
            pltpu.async_copy(b["widebuf"], num_sh.at[b["dscat"]], b["sem_s"],
                             add=True)

        jrow = jnp.full((16,), n_dst, jnp.int32)

        def wzero(p):
            b = parts(p)

            def wzbody(r, carry):
                for q in range(5):
                    b["widebuf"][r, pl.ds(q * 16, 16)] = zero16
                return carry

            lax.fori_loop(0, k_e, wzbody, 0)

            def dzbody(j, carry):
                b["dscat"][pl.ds(j * 16, 16)] = jrow
                return carry

            lax.fori_loop(0, k_e // 16, dzbody, 0)

        wzero(0)
        wzero(1)
        zc = min(k_e, r16)
        while r16 % zc:
            zc //= 2
        wb0 = parts(0)["widebuf"]

        def zbody(i, carry):
            pltpu.sync_copy(wb0.at[pl.ds(0, zc)],
                            num_sh.at[pl.ds(s * r16 + i * zc, zc)])
            return carry

        lax.fori_loop(0, r16 // zc, zbody, 0)
        pltpu.async_copy(parts(0)["widebuf"], num_sh.at[parts(0)["dscat"]],
                         parts(0)["sem_s"], add=True)
        pltpu.async_copy(parts(1)["widebuf"], num_sh.at[parts(1)["dscat"]],
                         parts(1)["sem_s"], add=True)
        if ssrc:
            pltpu.sync_copy(asrc_hbm, asrc_v)
        if sdst:
            pltpu.sync_copy(adst_hbm, adst_v)
        plsc.subcore_barrier()

        c0 = s * cw
        issue_idx(0, c0)
        wait_idx(0, c0)
        prep(0)
        issue_gat(0)

        def pair_body(i2, carry):
            a = c0 + 2 * i2
            nxt = jnp.minimum(a + 2, c0 + cw - 1)
            issue_idx(1, a + 1)
            wait_idx(1, a + 1)
            prep(1)
            issue_gat(1)
            wait_gat(0)
            scale_scatter(0)
            issue_idx(0, nxt)
            wait_idx(0, nxt)
            prep(0)
            issue_gat(0)
            wait_gat(1)
            scale_scatter(1)
            return carry

        lax.fori_loop(0, cw // 2, pair_body, 0)
        wait_gat(0)
        wait_scat(0)
        wait_scat(1)
        plsc.subcore_barrier()

        def drain_body(i, carry):
            r0 = s * r16 + i * cr
            pltpu.sync_copy(num_sh.at[pl.ds(r0, cr)], drainbuf)
            for r in range(cr):
                den = plsc.load_gather(
                    drainbuf,
                    [jnp.full((16,), r, jnp.int32),
                     jnp.full((16,), 64, jnp.int32)],
                )
                m = den > 0.0
                for q in range(4):
                    ql = pl.ds(q * 16, 16)
                    obuf[r, ql] = jnp.where(m, drainbuf[r, ql] / den, 0.0)
            pltpu.sync_copy(obuf, out_hbm.at[pl.ds(r0, cr), pl.ds(c * 64, 64)])
            return carry

        lax.fori_loop(0, r16 // cr, drain_body, 0)

    return k(h2, a_src, a_dst, src_e, dst_e)


def _readout_sc(hd, hc, drug1, drug2, cell):
    b_n = drug1.shape[0]
    rb = b_n // 32
    mesh = plsc.VectorSubcoreMesh(core_axis_name="c", subcore_axis_name="s")
    out_t = jax.ShapeDtypeStruct((b_n, 128), jnp.float32)

    @functools.partial(
        pl.kernel,
        out_type=(out_t, out_t, out_t),
        mesh=mesh,
        compiler_params=pltpu.CompilerParams(use_tc_tiling_on_sc=False, needs_layout_passes=False),
        scratch_types=[
            pltpu.VMEM((rb,), jnp.int32),
            pltpu.VMEM((rb, 128), jnp.float32),
            pltpu.SemaphoreType.DMA,
        ],
    )
    def k(hd_hbm, hc_hbm, d1_hbm, d2_hbm, cl_hbm, o1, o2, o3, idx_v, buf, sem):
        wid = lax.axis_index("s") * 2 + lax.axis_index("c")
        base = wid * rb
        for idx_hbm, tab_hbm, out_hbm in (
            (d1_hbm, hd_hbm, o1),
            (d2_hbm, hd_hbm, o2),
            (cl_hbm, hc_hbm, o3),
        ):
            pltpu.sync_copy(idx_hbm.at[pl.ds(base, rb)], idx_v)
            pltpu.async_copy(tab_hbm.at[idx_v], buf, sem).wait()
            pltpu.sync_copy(buf, out_hbm.at[pl.ds(base, rb)])

    return k(hd, hc, drug1, drug2, cell)


def _pad_edges(src, dst, n_dst, k_e):
    e = src.shape[0]
    e_pad = _round_up(e, _NS * k_e * 2)
    pad = e_pad - e
    src = jnp.concatenate([src, jnp.zeros((pad,), jnp.int32)])
    dst = jnp.concatenate([dst, jnp.full((pad,), n_dst, jnp.int32)])
    return src, dst


def _score_cols(w_l, specs):
    cols = [w_l[r] @ v[r] for (r, v) in specs]
    g = jnp.stack(cols, axis=1)
    return jnp.pad(g, ((0, 0), (0, 128 - g.shape[1])))


def _tail(plan):
    k_e, ssrc, sdst, n_pad = plan
    return (n_pad, k_e, ssrc, sdst)


def kernel(x_drug, x_protein, x_cell, edge_index_dd, edge_index_dp,
           edge_index_rev_dp, edge_index_pp, edge_index_cp, edge_index_rev_cp,
           drug1, drug2, cell, drug_table, protein_table, cell_table,
           W0, as0, ad0, b0, W1, as1, ad1, b1, cW1, cb1, cW2, cb2, cW3, cb3):
    nd = drug_table.shape[0]
    np_ = protein_table.shape[0]
    nc = cell_table.shape[0]
    pl_dd = _plan(nd, nd)
    pl_dp = _plan(nd, np_)
    pl_rdp = _plan(np_, nd)
    pl_pp = _plan(np_, np_)
    pl_cp = _plan(nc, np_)
    pl_rcp = _plan(np_, nc)
    pad_d = pl_dd[3]
    pad_p = pl_dp[3]
    pad_c = pl_rcp[3]

    hd0 = jnp.take(drug_table, x_drug, axis=0)
    hp0 = jnp.take(protein_table, x_protein, axis=0)
    hc0 = jnp.take(cell_table, x_cell, axis=0)

    ar_d = jnp.arange(nd, dtype=jnp.int32)
    ar_p = jnp.arange(np_, dtype=jnp.int32)
    s_dd, d_dd = _pad_edges(
        jnp.concatenate([edge_index_dd[0], ar_d]),
        jnp.concatenate([edge_index_dd[1], ar_d]), nd, pl_dd[0])
    s_dp, d_dp = _pad_edges(edge_index_dp[0], edge_index_dp[1], np_, pl_dp[0])
    s_rdp, d_rdp = _pad_edges(edge_index_rev_dp[0], edge_index_rev_dp[1], nd, pl_rdp[0])
    s_pp, d_pp = _pad_edges(
        jnp.concatenate([edge_index_pp[0], ar_p]),
        jnp.concatenate([edge_index_pp[1], ar_p]), np_, pl_pp[0])
    s_cp, d_cp = _pad_edges(edge_index_cp[0], edge_index_cp[1], np_, pl_cp[0])
    s_rcp, d_rcp = _pad_edges(edge_index_rev_cp[0], edge_index_rev_cp[1], nc, pl_rcp[0])

    zbias = jnp.zeros((1, 128), jnp.float32)

    gd0 = _score_cols(W0, [(0, as0), (1, as0), (0, ad0), (2, ad0)])
    gp0 = _score_cols(
        W0, [(2, as0), (3, as0), (5, as0), (1, ad0), (3, ad0), (4, ad0)])
    gc0 = _score_cols(W0, [(4, as0), (5, ad0)])
    yd = _proj_tc([hd0], zbias, jnp.stack([W0[0], W0[1], gd0]), nd, False)
    yp = _proj_tc([hp0], zbias,
                  jnp.stack([W0[2], W0[3], W0[5], gp0]), np_, False)
    yc = _proj_tc([hc0], zbias, jnp.stack([W0[4], gc0]), nc, False)

    sd = yd[2]
    sp = yp[3]
    sc = yc[1]

    def h2(y):
        return y.reshape(2 * y.shape[0], 64)

    od_dd = _gat_edge_sc(h2(yd[0]), sd[:, 0], sd[:, 2], s_dd, d_dd, nd, *_tail(pl_dd))
    op_dp = _gat_edge_sc(h2(yd[1]), sd[:, 1], sp[:, 3], s_dp, d_dp, np_, *_tail(pl_dp))
    od_rdp = _gat_edge_sc(
        h2(yp[0]), sp[:, 0], sd[:, 3], s_rdp, d_rdp, nd, *_tail(pl_rdp))
    op_pp = _gat_edge_sc(h2(yp[1]), sp[:, 1], sp[:, 4], s_pp, d_pp, np_, *_tail(pl_pp))
    op_cp = _gat_edge_sc(h2(yc[0]), sc[:, 0], sp[:, 5], s_cp, d_cp, np_, *_tail(pl_cp))
    oc_rcp = _gat_edge_sc(
        h2(yp[2]), sp[:, 2], sc[:, 1], s_rcp, d_rcp, nc, *_tail(pl_rcp))

    gd1 = _score_cols(W1, [(0, as1), (0, ad1), (2, ad1)])
    gp1 = _score_cols(W1, [(2, as1), (5, as1)])
    gc1 = _score_cols(W1, [(5, ad1)])
    bias_d = (b0[0] + b0[2]).reshape(1, 128)
    bias_p = (b0[1] + b0[3] + b0[4]).reshape(1, 128)
    bias_c = b0[5].reshape(1, 128)
    yd1 = _proj_tc([od_dd, od_rdp], bias_d, jnp.stack([W1[0], gd1]), nd, True)
    yp1 = _proj_tc([op_dp, op_pp, op_cp], bias_p,
                   jnp.stack([W1[2], W1[5], gp1]), np_, True)
    yc1 = _proj_tc([oc_rcp], bias_c, jnp.stack([gc1]), nc, True)

    sd1 = yd1[1]
    sp1 = yp1[2]
    sc1 = yc1[0]
    od_dd1 = _gat_edge_sc(
        h2(yd1[0]), sd1[:, 0], sd1[:, 1], s_dd, d_dd, nd, *_tail(pl_dd))
    od_rdp1 = _gat_edge_sc(
        h2(yp1[0]), sp1[:, 0], sd1[:, 2], s_rdp, d_rdp, nd, *_tail(pl_rdp))
    oc_rcp1 = _gat_edge_sc(
        h2(yp1[1]), sp1[:, 1], sc1[:, 0], s_rcp, d_rcp, nc, *_tail(pl_rcp))

    hd_fin = _finalize_tc(
        [od_dd1, od_rdp1], (b1[0] + b1[2]).reshape(1, 128), pad_d)
    hc_fin = _finalize_tc([oc_rcp1], b1[5].reshape(1, 128), pad_c)
    g1, g2, g3 = _readout_sc(hd_fin, hc_fin, drug1, drug2, cell)

    w3p = jnp.pad(cW3, ((0, 0), (0, 126)))
    b3p = jnp.pad(cb3, (0, 126)).reshape(1, 128)
    out = _mlp_tc(g1, g2, g3, cW1[:128], cW1[128:256], cW1[256:384],
                  cb1.reshape(1, 768), cW2, cb2.reshape(1, 256), w3p, b3p)
    return out[:, :2]
